# Initial kernel scaffold; baseline (speedup 1.0000x reference)
import jax
import jax.numpy as jnp
from jax.experimental import pallas as pl


def kernel(act, location, edge_index, batch, emb_act, emb_loc, W_gcn, b_gcn, W_fc, b_fc):
    raise NotImplementedError("write your pallas kernel here")


# speedup vs baseline: 15.1313x; 15.1313x over previous
import functools

import jax
import jax.numpy as jnp
from jax import lax
from jax.experimental import pallas as pl
from jax.experimental.pallas import tpu as pltpu
from jax.experimental.pallas import tpu_sc as plsc

N = 100000
E = 1600000
H = 32
G = 128
T = 10

L = 16
CH = 128
NS = 16
NFULL = N // CH
NTAIL = N - NFULL * CH
ECH = E // CH

f32 = jnp.float32
i32 = jnp.int32


def _nchunks_nodes(s):
  q, r = NFULL // NS, NFULL % NS
  return jnp.where(s < r, q + 1, q)


def _nchunks_edges(s):
  q, r = ECH // NS, ECH % NS
  return jnp.where(s < r, q + 1, q)


def _sc_embed_deg_body(act_h, loc_h, ei_h, batch_h, ea_h, el_h,
                       x_h, deg_h, cnt_h,
                       aidx, lidx, a32, l32, abuf, lbuf, xbuf,
                       onesv, ones32, zbufv,
                       deg_s, cnt_s, sem1, sem2):
  c = lax.axis_index("c")
  s = lax.axis_index("s")

  for i in range(CH // L):
    onesv[pl.ds(L * i, L)] = jnp.ones((L,), f32)
    zbufv[pl.ds(L * i, L)] = jnp.zeros((L,), f32)
  for i in range(NTAIL // L):
    ones32[pl.ds(L * i, L)] = jnp.ones((L,), f32)

  core0 = c == 0

  @pl.when(core0)
  def _():
    def zbody(k, carry):
      base = (s + NS * k) * CH
      pltpu.sync_copy(zbufv, deg_s.at[pl.ds(base, CH)])
      return carry
    lax.fori_loop(0, _nchunks_nodes(s), zbody, 0)

    @pl.when(s == NS - 1)
    def _():
      pltpu.sync_copy(zbufv.at[pl.ds(0, NTAIL)],
                      deg_s.at[pl.ds(NFULL * CH, NTAIL)])

    @pl.when(s == 0)
    def _():
      pltpu.sync_copy(zbufv, cnt_s)

  plsc.subcore_barrier()

  @pl.when(core0)
  def _():
    def dbody(k, carry):
      base = (s + NS * k) * CH
      pltpu.sync_copy(ei_h.at[1, pl.ds(base, CH)], aidx)
      pltpu.sync_copy(onesv, deg_s.at[aidx], add=True)
      return carry
    lax.fori_loop(0, _nchunks_edges(s), dbody, 0)

    def cbody(k, carry):
      base = (s + NS * k) * CH
      pltpu.sync_copy(batch_h.at[pl.ds(base, CH)], lidx)
      pltpu.sync_copy(onesv, cnt_s.at[lidx], add=True)
      return carry
    lax.fori_loop(0, _nchunks_nodes(s), cbody, 0)

    @pl.when(s == NS - 1)
    def _():
      pltpu.sync_copy(batch_h.at[pl.ds(NFULL * CH, NTAIL)], a32)
      pltpu.sync_copy(ones32, cnt_s.at[a32], add=True)

  @pl.when(c == 1)
  def _():
    def ebody(k, carry):
      base = (s + NS * k) * CH
      pltpu.sync_copy(act_h.at[pl.ds(base, CH)], aidx)
      pltpu.sync_copy(loc_h.at[pl.ds(base, CH)], lidx)
      cp1 = pltpu.async_copy(ea_h.at[aidx], abuf, sem1)
      cp2 = pltpu.async_copy(el_h.at[lidx], lbuf, sem2)
      cp1.wait()
      cp2.wait()

      def rbody(i, carry2):
        for j in range(H // L):
          v = abuf[i, pl.ds(L * j, L)] + lbuf[i, pl.ds(L * j, L)]
          xbuf[i, pl.ds(L * j, L)] = jnp.maximum(v, 0.0)
        return carry2
      lax.fori_loop(0, CH, rbody, 0)
      pltpu.sync_copy(xbuf, x_h.at[pl.ds(base, CH)])
      return carry
    lax.fori_loop(0, _nchunks_nodes(s), ebody, 0)

    @pl.when(s == NS - 1)
    def _():
      base = NFULL * CH
      pltpu.sync_copy(act_h.at[pl.ds(base, NTAIL)], a32)
      pltpu.sync_copy(loc_h.at[pl.ds(base, NTAIL)], l32)
      cp1 = pltpu.async_copy(ea_h.at[a32], abuf.at[pl.ds(0, NTAIL)], sem1)
      cp2 = pltpu.async_copy(el_h.at[l32], lbuf.at[pl.ds(0, NTAIL)], sem2)
      cp1.wait()
      cp2.wait()

      def rbody(i, carry2):
        for j in range(H // L):
          v = abuf[i, pl.ds(L * j, L)] + lbuf[i, pl.ds(L * j, L)]
          xbuf[i, pl.ds(L * j, L)] = jnp.maximum(v, 0.0)
        return carry2
      lax.fori_loop(0, NTAIL, rbody, 0)
      pltpu.sync_copy(xbuf.at[pl.ds(0, NTAIL)], x_h.at[pl.ds(base, NTAIL)])

  plsc.subcore_barrier()

  @pl.when(core0)
  def _():
    def wbody(k, carry):
      base = (s + NS * k) * CH
      pltpu.sync_copy(deg_s.at[pl.ds(base, CH)], deg_h.at[pl.ds(base, CH)])
      return carry
    lax.fori_loop(0, _nchunks_nodes(s), wbody, 0)

    @pl.when(s == NS - 1)
    def _():
      pltpu.sync_copy(deg_s.at[pl.ds(NFULL * CH, NTAIL)],
                      deg_h.at[pl.ds(NFULL * CH, NTAIL)])

    @pl.when(s == 0)
    def _():
      pltpu.sync_copy(cnt_s, cnt_h)


_sc_embed_deg = pl.kernel(
    _sc_embed_deg_body,
    out_type=(
        jax.ShapeDtypeStruct((N, H), f32),
        jax.ShapeDtypeStruct((N,), f32),
        jax.ShapeDtypeStruct((G,), f32),
    ),
    mesh=plsc.VectorSubcoreMesh(core_axis_name="c", subcore_axis_name="s"),
    compiler_params=pltpu.CompilerParams(use_tc_tiling_on_sc=False),
    scratch_types=[
        pltpu.VMEM((CH,), i32),
        pltpu.VMEM((CH,), i32),
        pltpu.VMEM((NTAIL,), i32),
        pltpu.VMEM((NTAIL,), i32),
        pltpu.VMEM((CH, H), f32),
        pltpu.VMEM((CH, H), f32),
        pltpu.VMEM((CH, H), f32),
        pltpu.VMEM((CH,), f32),
        pltpu.VMEM((NTAIL,), f32),
        pltpu.VMEM((CH,), f32),
        pltpu.VMEM_SHARED((N,), f32),
        pltpu.VMEM_SHARED((G,), f32),
        pltpu.SemaphoreType.DMA,
        pltpu.SemaphoreType.DMA,
    ],
)


_BN = 1000


def _tc_matmul_body(x_ref, deg_ref, w_ref, ystk_ref, dinvw_ref):
  xb = x_ref[...]
  w = w_ref[...]
  deg = deg_ref[...] + 1.0
  dinv = lax.rsqrt(deg)
  y = jnp.dot(xb, w, preferred_element_type=f32) * dinv
  ystk_ref[0] = y[:, :L]
  ystk_ref[1] = y[:, L:]
  dinvw_ref[...] = jnp.broadcast_to(dinv, (_BN, L))


def _tc_matmul(x, deg2, w):
  return pl.pallas_call(
      _tc_matmul_body,
      grid=(N // _BN,),
      in_specs=[
          pl.BlockSpec((_BN, H), lambda i: (i, 0)),
          pl.BlockSpec((_BN, 1), lambda i: (i, 0)),
          pl.BlockSpec((H, H), lambda i: (0, 0)),
      ],
      out_specs=[
          pl.BlockSpec((2, _BN, L), lambda i: (0, i, 0)),
          pl.BlockSpec((_BN, L), lambda i: (i, 0)),
      ],
      out_shape=[
          jax.ShapeDtypeStruct((2, N, L), f32),
          jax.ShapeDtypeStruct((N, L), f32),
      ],
  )(x, deg2, w)


def _sc_edge_pool_body(ystk_h, dinvw_h, ei_h, batch_h, b2_h,
                       out_h,
                       ridx, cidx, rbuf, abuf, ybuf, dbuf, hbuf,
                       btv, bt32, bvv, zb16,
                       acc_s, pooled_s, sem1):
  c = lax.axis_index("c")
  s = lax.axis_index("s")

  def zb_body(i, carry):
    zb16[i, :] = jnp.zeros((L,), f32)
    return carry
  lax.fori_loop(0, CH, zb_body, 0)

  def zacc(k, carry):
    base = (s + NS * k) * CH
    pltpu.sync_copy(zb16, acc_s.at[pl.ds(base, CH)])
    return carry
  lax.fori_loop(0, _nchunks_nodes(s), zacc, 0)

  @pl.when(s == NS - 1)
  def _():
    pltpu.sync_copy(zb16.at[pl.ds(0, NTAIL)],
                    acc_s.at[pl.ds(NFULL * CH, NTAIL)])

  @pl.when(s == 0)
  def _():
    pltpu.sync_copy(zb16, pooled_s)

  plsc.subcore_barrier()

  yhalf = ystk_h.at[c]

  def ebody(k, carry):
    base = (s + NS * k) * CH
    pltpu.sync_copy(ei_h.at[0, pl.ds(base, CH)], ridx)
    pltpu.sync_copy(ei_h.at[1, pl.ds(base, CH)], cidx)
    pltpu.async_copy(yhalf.at[ridx], rbuf, sem1).wait()
    pltpu.sync_copy(rbuf, acc_s.at[cidx], add=True)
    return carry
  lax.fori_loop(0, _nchunks_edges(s), ebody, 0)

  plsc.subcore_barrier()

  pltpu.sync_copy(b2_h.at[c], bvv)
  bv = bvv[...]

  def pbody(k, carry):
    base = (s + NS * k) * CH
    pltpu.sync_copy(acc_s.at[pl.ds(base, CH)], abuf)
    pltpu.sync_copy(ystk_h.at[c, pl.ds(base, CH)], ybuf)
    pltpu.sync_copy(dinvw_h.at[pl.ds(base, CH)], dbuf)
    pltpu.sync_copy(batch_h.at[pl.ds(base, CH)], btv)

    def hrow(i, carry2):
      v = (abuf[i, :] + ybuf[i, :]) * dbuf[i, :] + bv
      hbuf[i, :] = jnp.maximum(v, 0.0)
      return carry2
    lax.fori_loop(0, CH, hrow, 0)
    pltpu.sync_copy(hbuf, pooled_s.at[btv], add=True)
    return carry
  lax.fori_loop(0, _nchunks_nodes(s), pbody, 0)

  @pl.when(s == NS - 1)
  def _():
    base = NFULL * CH
    pltpu.sync_copy(acc_s.at[pl.ds(base, NTAIL)], abuf.at[pl.ds(0, NTAIL)])
    pltpu.sync_copy(ystk_h.at[c, pl.ds(base, NTAIL)],
                    ybuf.at[pl.ds(0, NTAIL)])
    pltpu.sync_copy(dinvw_h.at[pl.ds(base, NTAIL)], dbuf.at[pl.ds(0, NTAIL)])
    pltpu.sync_copy(batch_h.at[pl.ds(base, NTAIL)], bt32)

    def hrow(i, carry2):
      v = (abuf[i, :] + ybuf[i, :]) * dbuf[i, :] + bv
      hbuf[i, :] = jnp.maximum(v, 0.0)
      return carry2
    lax.fori_loop(0, NTAIL, hrow, 0)
    pltpu.sync_copy(hbuf.at[pl.ds(0, NTAIL)], pooled_s.at[bt32], add=True)

  plsc.subcore_barrier()

  @pl.when(s == 0)
  def _():
    pltpu.sync_copy(pooled_s, out_h.at[c])


_sc_edge_pool = pl.kernel(
    _sc_edge_pool_body,
    out_type=jax.ShapeDtypeStruct((2, G, L), f32),
    mesh=plsc.VectorSubcoreMesh(core_axis_name="c", subcore_axis_name="s"),
    compiler_params=pltpu.CompilerParams(use_tc_tiling_on_sc=False),
    scratch_types=[
        pltpu.VMEM((CH,), i32),
        pltpu.VMEM((CH,), i32),
        pltpu.VMEM((CH, L), f32),
        pltpu.VMEM((CH, L), f32),
        pltpu.VMEM((CH, L), f32),
        pltpu.VMEM((CH, L), f32),
        pltpu.VMEM((CH, L), f32),
        pltpu.VMEM((CH,), i32),
        pltpu.VMEM((NTAIL,), i32),
        pltpu.VMEM((L,), f32),
        pltpu.VMEM((CH, L), f32),
        pltpu.VMEM_SHARED((N, L), f32),
        pltpu.VMEM_SHARED((G, L), f32),
        pltpu.SemaphoreType.DMA,
    ],
)


def _tc_head_body(p_ref, cnt_ref, wfc_ref, bfc_ref, out_ref):
  ps = jnp.concatenate([p_ref[0], p_ref[1]], axis=1)
  cnt = jnp.maximum(cnt_ref[...], 1.0)
  pooled = ps / cnt
  logits = jnp.dot(pooled, wfc_ref[...], preferred_element_type=f32)
  logits = logits + bfc_ref[...]
  m = jnp.max(logits, axis=1, keepdims=True)
  e = jnp.exp(logits - m)
  lse = jnp.log(jnp.sum(e, axis=1, keepdims=True)) + m
  out_ref[...] = logits - lse


def _tc_head(pooled, cnt2, wfc, bfc2):
  return pl.pallas_call(
      _tc_head_body,
      out_shape=jax.ShapeDtypeStruct((G, T), f32),
  )(pooled, cnt2, wfc, bfc2)


def kernel(act, location, edge_index, batch, emb_act, emb_loc,
           W_gcn, b_gcn, W_fc, b_fc):
  act = act.astype(i32)
  location = location.astype(i32)
  edge_index = edge_index.astype(i32)
  batch = batch.astype(i32)

  x, deg, cnt = _sc_embed_deg(act, location, edge_index, batch,
                              emb_act, emb_loc)
  ystk, dinvw = _tc_matmul(x, deg.reshape(N, 1), W_gcn)
  pooled = _sc_edge_pool(ystk, dinvw, edge_index, batch,
                         b_gcn.reshape(2, L))
  return _tc_head(pooled, cnt.reshape(G, 1), W_fc, b_fc.reshape(1, T))


# speedup vs baseline: 27.3652x; 1.8085x over previous
import functools

import jax
import jax.numpy as jnp
from jax import lax
from jax.experimental import pallas as pl
from jax.experimental.pallas import tpu as pltpu
from jax.experimental.pallas import tpu_sc as plsc

N = 100000
E = 1600000
H = 32
G = 128
T = 10
VA = 1000
VL = 100

L = 16
CH = 128
NS = 16
NFULL = N // CH
NTAIL = N - NFULL * CH
ECH = E // CH
EB = 10
NBLK = ECH // EB

f32 = jnp.float32
i32 = jnp.int32


def _nchunks_nodes(s):
  q, r = NFULL // NS, NFULL % NS
  return jnp.where(s < r, q + 1, q)


def _nblocks_edges(s):
  q, r = NBLK // NS, NBLK % NS
  return jnp.where(s < r, q + 1, q)


def _sc_embed_deg_body(act_h, loc_h, ei3_h, batch_h, ea_h, el_h,
                       x_h, deg_h, cnt_h,
                       aidx, lidx, bidx, a32, l32, b32,
                       xbuf, onesv, ones32, zbufv, eav, elv,
                       deg_s, cnt_s, sem1):
  c = lax.axis_index("c")
  s = lax.axis_index("s")

  for i in range(CH // L):
    onesv[pl.ds(L * i, L)] = jnp.ones((L,), f32)
    zbufv[pl.ds(L * i, L)] = jnp.zeros((L,), f32)
  for i in range(NTAIL // L):
    ones32[pl.ds(L * i, L)] = jnp.ones((L,), f32)

  core0 = c == 0
  iotav = lax.iota(i32, L)

  @pl.when(core0)
  def _():
    def zbody(k, carry):
      base = (s + NS * k) * CH
      pltpu.sync_copy(zbufv, deg_s.at[pl.ds(base, CH)])
      return carry
    lax.fori_loop(0, _nchunks_nodes(s), zbody, 0)

    @pl.when(s == NS - 1)
    def _():
      pltpu.sync_copy(zbufv.at[pl.ds(0, NTAIL)],
                      deg_s.at[pl.ds(NFULL * CH, NTAIL)])

  @pl.when(jnp.logical_and(c == 1, s == 0))
  def _():
    pltpu.sync_copy(zbufv, cnt_s)

  plsc.subcore_barrier()

  @pl.when(core0)
  def _():
    def dbody(bk, carry):
      blk = (s + NS * bk) * EB
      pltpu.sync_copy(ei3_h.at[1, pl.ds(blk, EB)], bidx)
      for j in range(EB):
        pltpu.sync_copy(onesv, deg_s.at[bidx.at[j]], add=True)
      return carry
    lax.fori_loop(0, _nblocks_edges(s), dbody, 0)

  @pl.when(c == 1)
  def _():
    pltpu.sync_copy(ea_h, eav)
    pltpu.sync_copy(el_h, elv)

    def echunk(base, ngroups):
      for g in range(ngroups):
        acts = aidx[pl.ds(L * g, L)] * H
        locs = lidx[pl.ds(L * g, L)] * H
        lofs = (iotav + (L * g)) * H

        def fbody(f, carry2):
          fsp = jnp.full((L,), f, i32)
          va = plsc.load_gather(eav, [acts + fsp])
          vl = plsc.load_gather(elv, [locs + fsp])
          v = jnp.maximum(va + vl, 0.0)
          plsc.store_scatter(xbuf, [lofs + fsp], v)
          return carry2
        lax.fori_loop(0, H, fbody, 0)

    def ebody(k, carry):
      base = (s + NS * k) * CH
      pltpu.sync_copy(act_h.at[pl.ds(base, CH)], aidx)
      pltpu.sync_copy(loc_h.at[pl.ds(base, CH)], lidx)
      pltpu.sync_copy(batch_h.at[pl.ds(base, CH)], bidx2d0)
      echunk(base, CH // L)
      pltpu.sync_copy(xbuf, x_h.at[pl.ds(base * H, CH * H)])
      pltpu.sync_copy(onesv, cnt_s.at[bidx2d0], add=True)
      return carry
    bidx2d0 = bidx.at[0]
    lax.fori_loop(0, _nchunks_nodes(s), ebody, 0)

    @pl.when(s == NS - 1)
    def _():
      base = NFULL * CH
      pltpu.sync_copy(act_h.at[pl.ds(base, NTAIL)], a32)
      pltpu.sync_copy(loc_h.at[pl.ds(base, NTAIL)], l32)
      pltpu.sync_copy(batch_h.at[pl.ds(base, NTAIL)], b32)
      for g in range(NTAIL // L):
        acts = a32[pl.ds(L * g, L)] * H
        locs = l32[pl.ds(L * g, L)] * H
        lofs = (iotav + (L * g)) * H

        def fbody(f, carry2):
          fsp = jnp.full((L,), f, i32)
          va = plsc.load_gather(eav, [acts + fsp])
          vl = plsc.load_gather(elv, [locs + fsp])
          v = jnp.maximum(va + vl, 0.0)
          plsc.store_scatter(xbuf, [lofs + fsp], v)
          return carry2
        lax.fori_loop(0, H, fbody, 0)
      pltpu.sync_copy(xbuf.at[pl.ds(0, NTAIL * H)],
                      x_h.at[pl.ds(base * H, NTAIL * H)])
      pltpu.sync_copy(ones32, cnt_s.at[b32], add=True)

  plsc.subcore_barrier()

  WB = 6256
  @pl.when(core0)
  def _():
    @pl.when(s < NS - 1)
    def _():
      base = s * WB
      pltpu.sync_copy(deg_s.at[pl.ds(base, WB)], deg_h.at[pl.ds(base, WB)])

    @pl.when(s == NS - 1)
    def _():
      base = (NS - 1) * WB
      rem = N - base
      pltpu.sync_copy(deg_s.at[pl.ds(base, rem)], deg_h.at[pl.ds(base, rem)])

  @pl.when(jnp.logical_and(c == 1, s == 0))
  def _():
    pltpu.sync_copy(cnt_s, cnt_h)


_sc_embed_deg = pl.kernel(
    _sc_embed_deg_body,
    out_type=(
        jax.ShapeDtypeStruct((N * H,), f32),
        jax.ShapeDtypeStruct((N,), f32),
        jax.ShapeDtypeStruct((G,), f32),
    ),
    mesh=plsc.VectorSubcoreMesh(core_axis_name="c", subcore_axis_name="s"),
    compiler_params=pltpu.CompilerParams(use_tc_tiling_on_sc=False, needs_layout_passes=False),
    scratch_types=[
        pltpu.VMEM((CH,), i32),
        pltpu.VMEM((CH,), i32),
        pltpu.VMEM((EB, CH), i32),
        pltpu.VMEM((NTAIL,), i32),
        pltpu.VMEM((NTAIL,), i32),
        pltpu.VMEM((NTAIL,), i32),
        pltpu.VMEM((CH * H,), f32),
        pltpu.VMEM((CH,), f32),
        pltpu.VMEM((NTAIL,), f32),
        pltpu.VMEM((CH,), f32),
        pltpu.VMEM((VA * H,), f32),
        pltpu.VMEM((VL * H,), f32),
        pltpu.VMEM_SHARED((N,), f32),
        pltpu.VMEM_SHARED((G,), f32),
        pltpu.SemaphoreType.DMA,
    ],
)


_BN = 1000


def _tc_matmul_body(x_ref, deg_ref, w_ref, ystk_ref, dinvw_ref):
  xb = x_ref[...]
  w = w_ref[...]
  deg = deg_ref[...] + 1.0
  dinv = lax.rsqrt(deg)
  y = jnp.dot(xb, w, preferred_element_type=f32) * dinv
  ystk_ref[0] = y[:, :L]
  ystk_ref[1] = y[:, L:]
  dinvw_ref[...] = jnp.broadcast_to(dinv, (_BN, L))


def _tc_matmul(x, deg2, w):
  return pl.pallas_call(
      _tc_matmul_body,
      grid=(N // _BN,),
      in_specs=[
          pl.BlockSpec((_BN, H), lambda i: (i, 0)),
          pl.BlockSpec((_BN, 1), lambda i: (i, 0)),
          pl.BlockSpec((H, H), lambda i: (0, 0)),
      ],
      out_specs=[
          pl.BlockSpec((2, _BN, L), lambda i: (0, i, 0)),
          pl.BlockSpec((_BN, L), lambda i: (i, 0)),
      ],
      out_shape=[
          jax.ShapeDtypeStruct((2, N, L), f32),
          jax.ShapeDtypeStruct((N, L), f32),
      ],
  )(x, deg2, w)


def _sc_edge_pool_body(ystk_h, dinvw_h, ei3_h, batch_h, b2_h,
                       out_h,
                       ridx2, cidx2, rbuf0, rbuf1, abuf, ybuf, dbuf, hbuf,
                       btv, bt32, bvv, zb16,
                       acc_s, pooled_s, semg0, semg1, sema, semy, semd, semb):
  c = lax.axis_index("c")
  s = lax.axis_index("s")

  def zb_body(i, carry):
    zb16[i, :] = jnp.zeros((L,), f32)
    return carry
  lax.fori_loop(0, CH, zb_body, 0)

  def zacc(k, carry):
    base = (s + NS * k) * CH
    pltpu.sync_copy(zb16, acc_s.at[pl.ds(base, CH)])
    return carry
  lax.fori_loop(0, _nchunks_nodes(s), zacc, 0)

  @pl.when(s == NS - 1)
  def _():
    pltpu.sync_copy(zb16.at[pl.ds(0, NTAIL)],
                    acc_s.at[pl.ds(NFULL * CH, NTAIL)])

  @pl.when(s == 0)
  def _():
    pltpu.sync_copy(zb16, pooled_s)

  plsc.subcore_barrier()

  yhalf = ystk_h.at[c]
  bufs = (rbuf0, rbuf1)
  sems = (semg0, semg1)

  def ebody(bk, carry):
    blk = (s + NS * bk) * EB
    pltpu.sync_copy(ei3_h.at[0, pl.ds(blk, EB)], ridx2)
    pltpu.sync_copy(ei3_h.at[1, pl.ds(blk, EB)], cidx2)
    cps = [None] * EB
    cps[0] = pltpu.async_copy(yhalf.at[ridx2.at[0]], bufs[0], sems[0])
    for j in range(1, EB + 1):
      if j < EB:
        cps[j] = pltpu.async_copy(yhalf.at[ridx2.at[j]], bufs[j % 2],
                                  sems[j % 2])
      cps[j - 1].wait()
      pltpu.sync_copy(bufs[(j - 1) % 2], acc_s.at[cidx2.at[j - 1]], add=True)
    return carry
  lax.fori_loop(0, _nblocks_edges(s), ebody, 0)

  plsc.subcore_barrier()

  pltpu.sync_copy(b2_h.at[c], bvv)
  bv = bvv[...]

  def hcompute(nrows):
    def hrow(i, carry2):
      for u in range(4):
        ii = i * 4 + u
        v = (abuf[ii, :] + ybuf[ii, :]) * dbuf[ii, :] + bv
        hbuf[ii, :] = jnp.maximum(v, 0.0)
      return carry2
    lax.fori_loop(0, nrows // 4, hrow, 0)

  def pbody(k, carry):
    base = (s + NS * k) * CH
    cpa = pltpu.async_copy(acc_s.at[pl.ds(base, CH)], abuf, sema)
    cpy = pltpu.async_copy(ystk_h.at[c, pl.ds(base, CH)], ybuf, semy)
    cpd = pltpu.async_copy(dinvw_h.at[pl.ds(base, CH)], dbuf, semd)
    cpb = pltpu.async_copy(batch_h.at[pl.ds(base, CH)], btv, semb)
    cpa.wait()
    cpy.wait()
    cpd.wait()
    hcompute(CH)
    cpb.wait()
    pltpu.sync_copy(hbuf, pooled_s.at[btv], add=True)
    return carry
  lax.fori_loop(0, _nchunks_nodes(s), pbody, 0)

  @pl.when(s == NS - 1)
  def _():
    base = NFULL * CH
    pltpu.sync_copy(acc_s.at[pl.ds(base, NTAIL)], abuf.at[pl.ds(0, NTAIL)])
    pltpu.sync_copy(ystk_h.at[c, pl.ds(base, NTAIL)],
                    ybuf.at[pl.ds(0, NTAIL)])
    pltpu.sync_copy(dinvw_h.at[pl.ds(base, NTAIL)], dbuf.at[pl.ds(0, NTAIL)])
    pltpu.sync_copy(batch_h.at[pl.ds(base, NTAIL)], bt32)
    hcompute(NTAIL)
    pltpu.sync_copy(hbuf.at[pl.ds(0, NTAIL)], pooled_s.at[bt32], add=True)

  plsc.subcore_barrier()

  @pl.when(s == 0)
  def _():
    pltpu.sync_copy(pooled_s, out_h.at[c])


_sc_edge_pool = pl.kernel(
    _sc_edge_pool_body,
    out_type=jax.ShapeDtypeStruct((2, G, L), f32),
    mesh=plsc.VectorSubcoreMesh(core_axis_name="c", subcore_axis_name="s"),
    compiler_params=pltpu.CompilerParams(use_tc_tiling_on_sc=False, needs_layout_passes=False),
    scratch_types=[
        pltpu.VMEM((EB, CH), i32),
        pltpu.VMEM((EB, CH), i32),
        pltpu.VMEM((CH, L), f32),
        pltpu.VMEM((CH, L), f32),
        pltpu.VMEM((CH, L), f32),
        pltpu.VMEM((CH, L), f32),
        pltpu.VMEM((CH, L), f32),
        pltpu.VMEM((CH, L), f32),
        pltpu.VMEM((CH,), i32),
        pltpu.VMEM((NTAIL,), i32),
        pltpu.VMEM((L,), f32),
        pltpu.VMEM((CH, L), f32),
        pltpu.VMEM_SHARED((N, L), f32),
        pltpu.VMEM_SHARED((G, L), f32),
        pltpu.SemaphoreType.DMA,
        pltpu.SemaphoreType.DMA,
        pltpu.SemaphoreType.DMA,
        pltpu.SemaphoreType.DMA,
        pltpu.SemaphoreType.DMA,
        pltpu.SemaphoreType.DMA,
    ],
)


def _tc_head_body(p_ref, cnt_ref, wfc_ref, bfc_ref, out_ref):
  ps = jnp.concatenate([p_ref[0], p_ref[1]], axis=1)
  cnt = jnp.maximum(cnt_ref[...], 1.0)
  pooled = ps / cnt
  logits = jnp.dot(pooled, wfc_ref[...], preferred_element_type=f32)
  logits = logits + bfc_ref[...]
  m = jnp.max(logits, axis=1, keepdims=True)
  e = jnp.exp(logits - m)
  lse = jnp.log(jnp.sum(e, axis=1, keepdims=True)) + m
  out_ref[...] = logits - lse


def _tc_head(pooled, cnt2, wfc, bfc2):
  return pl.pallas_call(
      _tc_head_body,
      out_shape=jax.ShapeDtypeStruct((G, T), f32),
  )(pooled, cnt2, wfc, bfc2)


def kernel(act, location, edge_index, batch, emb_act, emb_loc,
           W_gcn, b_gcn, W_fc, b_fc):
  act = act.astype(i32)
  location = location.astype(i32)
  edge_index = edge_index.astype(i32)
  batch = batch.astype(i32)
  ei3 = edge_index.reshape(2, ECH, CH)

  xf, deg, cnt = _sc_embed_deg(act, location, ei3, batch,
                               emb_act.reshape(VA * H), emb_loc.reshape(VL * H))
  ystk, dinvw = _tc_matmul(xf.reshape(N, H), deg.reshape(N, 1), W_gcn)
  pooled = _sc_edge_pool(ystk, dinvw, ei3, batch, b_gcn.reshape(2, L))
  return _tc_head(pooled, cnt.reshape(G, 1), W_fc, b_fc.reshape(1, T))


# speedup vs baseline: 34.1632x; 1.2484x over previous
import jax
import jax.numpy as jnp
from jax import lax
from jax.experimental import pallas as pl
from jax.experimental.pallas import tpu as pltpu
from jax.experimental.pallas import tpu_sc as plsc

N = 100000
E = 1600000
H = 32
G = 128
T = 10
VA = 1000
VL = 100

L = 16
CH = 128
NS = 16
NW = 32
NFULL = N // CH
NTAIL = N - NFULL * CH
ECH = E // CH
EB = 10
NBLK = ECH // EB

f32 = jnp.float32
i32 = jnp.int32


def _split(total, nworkers, w):
  q, r = total // nworkers, total % nworkers
  return jnp.where(w < r, q + 1, q)


def _sc_embed_deg_body(act_h, loc_h, ei3_h, batch_h, ea_h, el_h,
                       x_h, deg0_h, deg1_h, cnt_h,
                       aidx, lidx, bidx, a32, l32, b32,
                       xbuf, onesv, ones32, zbufv, eav, elv,
                       deg_s, cnt_s, sem1):
  c = lax.axis_index("c")
  s = lax.axis_index("s")
  w = c * NS + s

  for i in range(CH // L):
    onesv[pl.ds(L * i, L)] = jnp.ones((L,), f32)
    zbufv[pl.ds(L * i, L)] = jnp.zeros((L,), f32)
  for i in range(NTAIL // L):
    ones32[pl.ds(L * i, L)] = jnp.ones((L,), f32)

  iotav = lax.iota(i32, L)

  def zbody(k, carry):
    base = (s + NS * k) * CH
    pltpu.sync_copy(zbufv, deg_s.at[pl.ds(base, CH)])
    return carry
  lax.fori_loop(0, _split(NFULL, NS, s), zbody, 0)

  @pl.when(s == NS - 1)
  def _():
    pltpu.sync_copy(zbufv.at[pl.ds(0, NTAIL)],
                    deg_s.at[pl.ds(NFULL * CH, NTAIL)])

  @pl.when(s == 0)
  def _():
    pltpu.sync_copy(zbufv, cnt_s)

  pltpu.sync_copy(ea_h, eav)
  pltpu.sync_copy(el_h, elv)

  plsc.subcore_barrier()

  def dbody(bk, carry):
    blk = (w + NW * bk) * EB
    pltpu.sync_copy(ei3_h.at[1, pl.ds(blk, EB)], bidx)
    for j in range(EB):
      pltpu.sync_copy(onesv, deg_s.at[bidx.at[j]], add=True)
    return carry
  lax.fori_loop(0, _split(NBLK, NW, w), dbody, 0)

  def echunk(aref, lref, ngroups):
    for g in range(ngroups):
      acts = aref[pl.ds(L * g, L)] * H
      locs = lref[pl.ds(L * g, L)] * H
      lofs = (iotav + (L * g)) * H

      def f4body(f4, carry2):
        for u in range(4):
          f = f4 * 4 + u
          fsp = jnp.full((L,), f, i32)
          va = plsc.load_gather(eav, [acts + fsp])
          vl = plsc.load_gather(elv, [locs + fsp])
          v = jnp.maximum(va + vl, 0.0)
          plsc.store_scatter(xbuf, [lofs + fsp], v)
        return carry2
      lax.fori_loop(0, H // 4, f4body, 0)

  bidx0 = bidx.at[0]

  def ebody(k, carry):
    base = (w + NW * k) * CH
    pltpu.sync_copy(act_h.at[pl.ds(base, CH)], aidx)
    pltpu.sync_copy(loc_h.at[pl.ds(base, CH)], lidx)
    pltpu.sync_copy(batch_h.at[pl.ds(base, CH)], bidx0)
    echunk(aidx, lidx, CH // L)
    pltpu.sync_copy(xbuf, x_h.at[pl.ds(base * H, CH * H)])
    pltpu.sync_copy(onesv, cnt_s.at[bidx0], add=True)
    return carry
  lax.fori_loop(0, _split(NFULL, NW, w), ebody, 0)

  @pl.when(w == NW - 1)
  def _():
    base = NFULL * CH
    pltpu.sync_copy(act_h.at[pl.ds(base, NTAIL)], a32)
    pltpu.sync_copy(loc_h.at[pl.ds(base, NTAIL)], l32)
    pltpu.sync_copy(batch_h.at[pl.ds(base, NTAIL)], b32)
    echunk(a32, l32, NTAIL // L)
    pltpu.sync_copy(xbuf.at[pl.ds(0, NTAIL * H)],
                    x_h.at[pl.ds(base * H, NTAIL * H)])
    pltpu.sync_copy(ones32, cnt_s.at[b32], add=True)

  plsc.subcore_barrier()

  WB = 6256

  def wb_deg(dst):
    @pl.when(s < NS - 1)
    def _():
      base = s * WB
      pltpu.sync_copy(deg_s.at[pl.ds(base, WB)], dst.at[pl.ds(base, WB)])

    @pl.when(s == NS - 1)
    def _():
      base = (NS - 1) * WB
      rem = N - base
      pltpu.sync_copy(deg_s.at[pl.ds(base, rem)], dst.at[pl.ds(base, rem)])

  @pl.when(c == 0)
  def _():
    wb_deg(deg0_h)

  @pl.when(c == 1)
  def _():
    wb_deg(deg1_h)

  @pl.when(s == 0)
  def _():
    pltpu.sync_copy(cnt_s, cnt_h.at[c])


_sc_embed_deg = pl.kernel(
    _sc_embed_deg_body,
    out_type=(
        jax.ShapeDtypeStruct((N * H,), f32),
        jax.ShapeDtypeStruct((N,), f32),
        jax.ShapeDtypeStruct((N,), f32),
        jax.ShapeDtypeStruct((2, G), f32),
    ),
    mesh=plsc.VectorSubcoreMesh(core_axis_name="c", subcore_axis_name="s"),
    compiler_params=pltpu.CompilerParams(use_tc_tiling_on_sc=False,
                                         needs_layout_passes=False),
    scratch_types=[
        pltpu.VMEM((CH,), i32),
        pltpu.VMEM((CH,), i32),
        pltpu.VMEM((EB, CH), i32),
        pltpu.VMEM((NTAIL,), i32),
        pltpu.VMEM((NTAIL,), i32),
        pltpu.VMEM((NTAIL,), i32),
        pltpu.VMEM((CH * H,), f32),
        pltpu.VMEM((CH,), f32),
        pltpu.VMEM((NTAIL,), f32),
        pltpu.VMEM((CH,), f32),
        pltpu.VMEM((VA * H,), f32),
        pltpu.VMEM((VL * H,), f32),
        pltpu.VMEM_SHARED((N,), f32),
        pltpu.VMEM_SHARED((G,), f32),
        pltpu.SemaphoreType.DMA,
    ],
)


_BN = 1000


def _tc_matmul_body(x_ref, d0_ref, d1_ref, w_ref, ystk_ref, dinvw_ref):
  xb = x_ref[...]
  w = w_ref[...]
  deg = d0_ref[...] + d1_ref[...] + 1.0
  dinv = lax.rsqrt(deg)
  y = jnp.dot(xb, w, preferred_element_type=f32) * dinv
  ystk_ref[0] = y[:, :L]
  ystk_ref[1] = y[:, L:]
  dinvw_ref[...] = jnp.broadcast_to(dinv, (_BN, L))


def _tc_matmul(x, deg0, deg1, w):
  return pl.pallas_call(
      _tc_matmul_body,
      grid=(N // _BN,),
      in_specs=[
          pl.BlockSpec((_BN, H), lambda i: (i, 0)),
          pl.BlockSpec((_BN, 1), lambda i: (i, 0)),
          pl.BlockSpec((_BN, 1), lambda i: (i, 0)),
          pl.BlockSpec((H, H), lambda i: (0, 0)),
      ],
      out_specs=[
          pl.BlockSpec((2, _BN, L), lambda i: (0, i, 0)),
          pl.BlockSpec((_BN, L), lambda i: (i, 0)),
      ],
      out_shape=[
          jax.ShapeDtypeStruct((2, N, L), f32),
          jax.ShapeDtypeStruct((N, L), f32),
      ],
  )(x, deg0, deg1, w)


_NRING = 8


def _sc_edge_pool_body(ystk_h, dinvw_h, ei3_h, batch_h, b2_h,
                       out_h,
                       ridx2, cidx2,
                       g0, g1, g2, g3, g4, g5, g6, g7,
                       abuf, ybuf, dbuf, hbuf,
                       btv, bt32, bvv,
                       acc_s, pooled_s,
                       semg, semsc, sema, semy, semd, semb):
  c = lax.axis_index("c")
  s = lax.axis_index("s")
  gbufs = (g0, g1, g2, g3, g4, g5, g6, g7)

  def zb_body(i, carry):
    hbuf[i, :] = jnp.zeros((L,), f32)
    return carry
  lax.fori_loop(0, CH, zb_body, 0)

  def zacc(k, carry):
    base = (s + NS * k) * CH
    pltpu.sync_copy(hbuf, acc_s.at[pl.ds(base, CH)])
    return carry
  lax.fori_loop(0, _split(NFULL, NS, s), zacc, 0)

  @pl.when(s == NS - 1)
  def _():
    pltpu.sync_copy(hbuf.at[pl.ds(0, NTAIL)],
                    acc_s.at[pl.ds(NFULL * CH, NTAIL)])

  @pl.when(s == 0)
  def _():
    pltpu.sync_copy(hbuf, pooled_s)

  plsc.subcore_barrier()

  yhalf = ystk_h.at[c]

  def ebody(bk, carry):
    blk = (s + NS * bk) * EB
    pltpu.sync_copy(ei3_h.at[0, pl.ds(blk, EB)], ridx2)
    pltpu.sync_copy(ei3_h.at[1, pl.ds(blk, EB)], cidx2)
    gcps = [None] * EB
    for j in range(_NRING):
      gcps[j] = pltpu.async_copy(yhalf.at[ridx2.at[j]], gbufs[j], semg)
    scps = [None] * EB
    for j in range(EB):
      gcps[j].wait()
      scps[j] = pltpu.async_copy(gbufs[j % _NRING], acc_s.at[cidx2.at[j]],
                                 semsc, add=True)
      if j + _NRING < EB:
        scps[j].wait()
        gcps[j + _NRING] = pltpu.async_copy(yhalf.at[ridx2.at[j + _NRING]],
                                            gbufs[j % _NRING], semg)
    for j in range(EB - _NRING, EB):
      scps[j].wait()
    return carry
  lax.fori_loop(0, _split(NBLK, NS, s), ebody, 0)

  plsc.subcore_barrier()

  pltpu.sync_copy(b2_h.at[c], bvv)
  bv = bvv[...]

  def hcompute(nrows):
    def hrow(i, carry2):
      for u in range(4):
        ii = i * 4 + u
        v = (abuf[ii, :] + ybuf[ii, :]) * dbuf[ii, :] + bv
        hbuf[ii, :] = jnp.maximum(v, 0.0)
      return carry2
    lax.fori_loop(0, nrows // 4, hrow, 0)

  def pbody(k, carry):
    base = (s + NS * k) * CH
    cpa = pltpu.async_copy(acc_s.at[pl.ds(base, CH)], abuf, sema)
    cpy = pltpu.async_copy(ystk_h.at[c, pl.ds(base, CH)], ybuf, semy)
    cpd = pltpu.async_copy(dinvw_h.at[pl.ds(base, CH)], dbuf, semd)
    cpb = pltpu.async_copy(batch_h.at[pl.ds(base, CH)], btv, semb)
    cpa.wait()
    cpy.wait()
    cpd.wait()
    hcompute(CH)
    cpb.wait()
    pltpu.sync_copy(hbuf, pooled_s.at[btv], add=True)
    return carry
  lax.fori_loop(0, _split(NFULL, NS, s), pbody, 0)

  @pl.when(s == NS - 1)
  def _():
    base = NFULL * CH
    pltpu.sync_copy(acc_s.at[pl.ds(base, NTAIL)], abuf.at[pl.ds(0, NTAIL)])
    pltpu.sync_copy(ystk_h.at[c, pl.ds(base, NTAIL)],
                    ybuf.at[pl.ds(0, NTAIL)])
    pltpu.sync_copy(dinvw_h.at[pl.ds(base, NTAIL)], dbuf.at[pl.ds(0, NTAIL)])
    pltpu.sync_copy(batch_h.at[pl.ds(base, NTAIL)], bt32)
    hcompute(NTAIL)
    pltpu.sync_copy(hbuf.at[pl.ds(0, NTAIL)], pooled_s.at[bt32], add=True)

  plsc.subcore_barrier()

  @pl.when(s == 0)
  def _():
    pltpu.sync_copy(pooled_s, out_h.at[c])


_sc_edge_pool = pl.kernel(
    _sc_edge_pool_body,
    out_type=jax.ShapeDtypeStruct((2, G, L), f32),
    mesh=plsc.VectorSubcoreMesh(core_axis_name="c", subcore_axis_name="s"),
    compiler_params=pltpu.CompilerParams(use_tc_tiling_on_sc=False,
                                         needs_layout_passes=False),
    scratch_types=[
        pltpu.VMEM((EB, CH), i32),
        pltpu.VMEM((EB, CH), i32),
    ] + [pltpu.VMEM((CH, L), f32)] * _NRING + [
        pltpu.VMEM((CH, L), f32),
        pltpu.VMEM((CH, L), f32),
        pltpu.VMEM((CH, L), f32),
        pltpu.VMEM((CH, L), f32),
        pltpu.VMEM((CH,), i32),
        pltpu.VMEM((NTAIL,), i32),
        pltpu.VMEM((L,), f32),
        pltpu.VMEM_SHARED((N, L), f32),
        pltpu.VMEM_SHARED((G, L), f32),
        pltpu.SemaphoreType.DMA,
        pltpu.SemaphoreType.DMA,
        pltpu.SemaphoreType.DMA,
        pltpu.SemaphoreType.DMA,
        pltpu.SemaphoreType.DMA,
        pltpu.SemaphoreType.DMA,
    ],
)


def _tc_head_body(p_ref, cnt_ref, wfc_ref, bfc_ref, out_ref):
  ps = jnp.concatenate([p_ref[0], p_ref[1]], axis=1)
  cnt = jnp.maximum(cnt_ref[0] + cnt_ref[1], 1.0)
  pooled = ps / cnt
  logits = jnp.dot(pooled, wfc_ref[...], preferred_element_type=f32)
  logits = logits + bfc_ref[...]
  m = jnp.max(logits, axis=1, keepdims=True)
  e = jnp.exp(logits - m)
  lse = jnp.log(jnp.sum(e, axis=1, keepdims=True)) + m
  out_ref[...] = logits - lse


def _tc_head(pooled, cnt3, wfc, bfc2):
  return pl.pallas_call(
      _tc_head_body,
      out_shape=jax.ShapeDtypeStruct((G, T), f32),
  )(pooled, cnt3, wfc, bfc2)


def kernel(act, location, edge_index, batch, emb_act, emb_loc,
           W_gcn, b_gcn, W_fc, b_fc):
  act = act.astype(i32)
  location = location.astype(i32)
  edge_index = edge_index.astype(i32)
  batch = batch.astype(i32)
  ei3 = edge_index.reshape(2, ECH, CH)

  xf, deg0, deg1, cnt = _sc_embed_deg(act, location, ei3, batch,
                                      emb_act.reshape(VA * H),
                                      emb_loc.reshape(VL * H))
  ystk, dinvw = _tc_matmul(xf.reshape(N, H), deg0.reshape(N, 1),
                           deg1.reshape(N, 1), W_gcn)
  pooled = _sc_edge_pool(ystk, dinvw, ei3, batch, b_gcn.reshape(2, L))
  return _tc_head(pooled, cnt.reshape(2, G, 1), W_fc, b_fc.reshape(1, T))


# speedup vs baseline: 45.1940x; 1.3229x over previous
import jax
import jax.numpy as jnp
from jax import lax
from jax.experimental import pallas as pl
from jax.experimental.pallas import tpu as pltpu
from jax.experimental.pallas import tpu_sc as plsc

N = 100000
E = 1600000
H = 32
G = 128
T = 10
VA = 1000
VL = 100

L = 16
CH = 128
NS = 16
NW = 32
NFULL = N // CH
NTAIL = N - NFULL * CH
ECH = E // CH
EB = 10
NBLK = ECH // EB
XROWS = N * H // 128

f32 = jnp.float32
i32 = jnp.int32


def _split(total, nworkers, w):
  q, r = total // nworkers, total % nworkers
  return jnp.where(w < r, q + 1, q)


def _rsqrt_nr(d):
  u = plsc.bitcast(d, i32)
  u = jnp.int32(0x5F3759DF) - lax.shift_right_logical(u, 1)
  t = plsc.bitcast(u, f32)
  for _ in range(3):
    t = t * (1.5 - 0.5 * d * t * t)
  return t


def _sc_embed_deg_body(act_h, loc_h, ei3_h, batch_h, ea_h, el_h,
                       x2_h, deg_h, cnt_h,
                       aidx, lidx, bidx, a32, l32, b32, degv,
                       xbuf2, onesv, ones32, zbufv, eav, elv,
                       deg_s, cnt_s, semsc):
  c = lax.axis_index("c")
  s = lax.axis_index("s")
  w = c * NS + s

  for i in range(CH // L):
    onesv[pl.ds(L * i, L)] = jnp.ones((L,), f32)
    zbufv[pl.ds(L * i, L)] = jnp.zeros((L,), f32)
  for i in range(NTAIL // L):
    ones32[pl.ds(L * i, L)] = jnp.ones((L,), f32)

  iotav = lax.iota(i32, L)

  def zbody(k, carry):
    base = (s + NS * k) * CH
    pltpu.sync_copy(zbufv, deg_s.at[pl.ds(base, CH)])
    return carry
  lax.fori_loop(0, _split(NFULL, NS, s), zbody, 0)

  @pl.when(s == NS - 1)
  def _():
    pltpu.sync_copy(zbufv.at[pl.ds(0, NTAIL)],
                    deg_s.at[pl.ds(NFULL * CH, NTAIL)])

  @pl.when(s == 0)
  def _():
    pltpu.sync_copy(zbufv, cnt_s)

  pltpu.sync_copy(ea_h, eav)
  pltpu.sync_copy(el_h, elv)

  plsc.subcore_barrier()

  def dbody(bk, carry):
    blk = (s + NS * bk) * EB
    pltpu.sync_copy(ei3_h.at[1, pl.ds(blk, EB)], bidx)
    scps = [pltpu.async_copy(onesv, deg_s.at[bidx.at[j]], semsc, add=True)
            for j in range(EB)]
    for cp in scps:
      cp.wait()
    return carry
  lax.fori_loop(0, _split(NBLK, NS, s), dbody, 0)

  plsc.subcore_barrier()

  def echunk(aref, lref, ngroups):
    for g in range(ngroups):
      acts = aref[pl.ds(L * g, L)] * H
      locs = lref[pl.ds(L * g, L)] * H
      lofs = (iotav + (L * g)) * H
      dv = _rsqrt_nr(degv[pl.ds(L * g, L)] + 1.0)

      def f4body(f4, carry2):
        for u in range(4):
          f = f4 * 4 + u
          fsp = jnp.full((L,), f, i32)
          va = plsc.load_gather(eav, [acts + fsp])
          vl = plsc.load_gather(elv, [locs + fsp])
          v = jnp.maximum(va + vl, 0.0) * dv
          flat = lofs + fsp
          plsc.store_scatter(
              xbuf2,
              [lax.shift_right_logical(flat, 7),
               jnp.bitwise_and(flat, 127)], v)
        return carry2
      lax.fori_loop(0, H // 4, f4body, 0)

  bidx0 = bidx.at[0]

  def ebody(k, carry):
    base = (w + NW * k) * CH
    pltpu.sync_copy(act_h.at[pl.ds(base, CH)], aidx)
    pltpu.sync_copy(loc_h.at[pl.ds(base, CH)], lidx)
    pltpu.sync_copy(batch_h.at[pl.ds(base, CH)], bidx0)
    pltpu.sync_copy(deg_s.at[pl.ds(base, CH)], degv)
    echunk(aidx, lidx, CH // L)
    pltpu.sync_copy(xbuf2, x2_h.at[pl.ds(base // 4, CH * H // 128)])
    pltpu.sync_copy(onesv, cnt_s.at[bidx0], add=True)
    return carry
  lax.fori_loop(0, _split(NFULL, NW, w), ebody, 0)

  @pl.when(w == NW - 1)
  def _():
    base = NFULL * CH
    pltpu.sync_copy(act_h.at[pl.ds(base, NTAIL)], a32)
    pltpu.sync_copy(loc_h.at[pl.ds(base, NTAIL)], l32)
    pltpu.sync_copy(batch_h.at[pl.ds(base, NTAIL)], b32)
    pltpu.sync_copy(deg_s.at[pl.ds(base, NTAIL)], degv.at[pl.ds(0, NTAIL)])
    echunk(a32, l32, NTAIL // L)
    pltpu.sync_copy(xbuf2.at[pl.ds(0, NTAIL * H // 128)],
                    x2_h.at[pl.ds(base // 4, NTAIL * H // 128)])
    pltpu.sync_copy(ones32, cnt_s.at[b32], add=True)

  plsc.subcore_barrier()

  WB = 6256

  @pl.when(c == 0)
  def _():
    @pl.when(s < NS - 1)
    def _():
      base = s * WB
      pltpu.sync_copy(deg_s.at[pl.ds(base, WB)], deg_h.at[pl.ds(base, WB)])

    @pl.when(s == NS - 1)
    def _():
      base = (NS - 1) * WB
      rem = N - base
      pltpu.sync_copy(deg_s.at[pl.ds(base, rem)], deg_h.at[pl.ds(base, rem)])

  @pl.when(s == 0)
  def _():
    pltpu.sync_copy(cnt_s, cnt_h.at[c])


_sc_embed_deg = pl.kernel(
    _sc_embed_deg_body,
    out_type=(
        jax.ShapeDtypeStruct((XROWS, 128), f32),
        jax.ShapeDtypeStruct((N,), f32),
        jax.ShapeDtypeStruct((2, G), f32),
    ),
    mesh=plsc.VectorSubcoreMesh(core_axis_name="c", subcore_axis_name="s"),
    compiler_params=pltpu.CompilerParams(use_tc_tiling_on_sc=False,
                                         needs_layout_passes=False),
    scratch_types=[
        pltpu.VMEM((CH,), i32),
        pltpu.VMEM((CH,), i32),
        pltpu.VMEM((EB, CH), i32),
        pltpu.VMEM((NTAIL,), i32),
        pltpu.VMEM((NTAIL,), i32),
        pltpu.VMEM((NTAIL,), i32),
        pltpu.VMEM((CH,), f32),
        pltpu.VMEM((CH * H // 128, 128), f32),
        pltpu.VMEM((CH,), f32),
        pltpu.VMEM((NTAIL,), f32),
        pltpu.VMEM((CH,), f32),
        pltpu.VMEM((VA * H,), f32),
        pltpu.VMEM((VL * H,), f32),
        pltpu.VMEM_SHARED((N,), f32),
        pltpu.VMEM_SHARED((G,), f32),
        pltpu.SemaphoreType.DMA,
    ],
)


_BX = 5000


def _tc_matmul_body(x_ref, w_ref, y_ref):
  y_ref[...] = jnp.dot(x_ref[...], w_ref[...], preferred_element_type=f32)


def _tc_matmul(x2, wbd):
  return pl.pallas_call(
      _tc_matmul_body,
      grid=(XROWS // _BX,),
      in_specs=[
          pl.BlockSpec((_BX, 128), lambda i: (i, 0)),
          pl.BlockSpec((128, 128), lambda i: (0, 0)),
      ],
      out_specs=pl.BlockSpec((_BX, 128), lambda i: (i, 0)),
      out_shape=jax.ShapeDtypeStruct((XROWS, 128), f32),
  )(x2, wbd)


_NRING = 8


def _sc_edge_pool_body(y2_h, deg_h, ei3_h, batch_h, b2_h,
                       out_h,
                       ridx2, cidx2, gidx2, eidx,
                       g0, g1, g2, g3, g4, g5, g6, g7,
                       abuf, ybuf, hbuf,
                       btv, bt32, bvv, degv, dinvv,
                       acc_s, pooled_s,
                       semg, semsc, sema, semy, semb):
  c = lax.axis_index("c")
  s = lax.axis_index("s")
  gbufs = (g0, g1, g2, g3, g4, g5, g6, g7)
  iotav = lax.iota(i32, L)

  def zb_body(i, carry):
    hbuf[i, :] = jnp.zeros((L,), f32)
    return carry
  lax.fori_loop(0, CH, zb_body, 0)

  def zacc(k, carry):
    base = (s + NS * k) * CH
    pltpu.sync_copy(hbuf, acc_s.at[pl.ds(base, CH)])
    return carry
  lax.fori_loop(0, _split(NFULL, NS, s), zacc, 0)

  @pl.when(s == NS - 1)
  def _():
    pltpu.sync_copy(hbuf.at[pl.ds(0, NTAIL)],
                    acc_s.at[pl.ds(NFULL * CH, NTAIL)])

  @pl.when(s == 0)
  def _():
    pltpu.sync_copy(hbuf, pooled_s)

  plsc.subcore_barrier()

  def ebody(bk, carry):
    blk = (s + NS * bk) * EB
    pltpu.sync_copy(ei3_h.at[0, pl.ds(blk, EB)], ridx2)
    pltpu.sync_copy(ei3_h.at[1, pl.ds(blk, EB)], cidx2)
    for j in range(EB):
      for t in range(CH // L):
        v = ridx2[j, pl.ds(L * t, L)]
        gidx2[j, pl.ds(L * t, L)] = v + v + c
    gcps = [None] * EB
    for j in range(_NRING):
      gcps[j] = pltpu.async_copy(y2_h.at[gidx2.at[j]], gbufs[j], semg)
    scps = [None] * EB
    for j in range(EB):
      gcps[j].wait()
      scps[j] = pltpu.async_copy(gbufs[j % _NRING], acc_s.at[cidx2.at[j]],
                                 semsc, add=True)
      if j + _NRING < EB:
        scps[j].wait()
        gcps[j + _NRING] = pltpu.async_copy(y2_h.at[gidx2.at[j + _NRING]],
                                            gbufs[j % _NRING], semg)
    for j in range(EB - _NRING, EB):
      scps[j].wait()
    return carry
  lax.fori_loop(0, _split(NBLK, NS, s), ebody, 0)

  plsc.subcore_barrier()

  pltpu.sync_copy(b2_h.at[c], bvv)
  bv = bvv[...]

  def prep_dinv(nrows):
    for t in range(nrows // L):
      d = degv[pl.ds(L * t, L)] + 1.0
      dinvv[pl.ds(L * t, L)] = _rsqrt_nr(d)

  def build_eidx(base, nrows):
    for t in range(nrows // L):
      v = base + iotav + (L * t)
      eidx[pl.ds(L * t, L)] = v + v + c

  def hcompute(nrows):
    def hrow(i, carry2):
      for u in range(4):
        ii = i * 4 + u
        dsp = plsc.load_gather(dinvv, [jnp.full((L,), ii, i32)])
        v = (abuf[ii, :] + ybuf[ii, :]) * dsp + bv
        hbuf[ii, :] = jnp.maximum(v, 0.0)
      return carry2
    lax.fori_loop(0, nrows // 4, hrow, 0)

  def pbody(k, carry):
    base = (s + NS * k) * CH
    build_eidx(base, CH)
    cpa = pltpu.async_copy(acc_s.at[pl.ds(base, CH)], abuf, sema)
    cpy = pltpu.async_copy(y2_h.at[eidx], ybuf, semy)
    cpd = pltpu.async_copy(deg_h.at[pl.ds(base, CH)], degv, semsc)
    cpb = pltpu.async_copy(batch_h.at[pl.ds(base, CH)], btv, semb)
    cpd.wait()
    prep_dinv(CH)
    cpa.wait()
    cpy.wait()
    hcompute(CH)
    cpb.wait()
    pltpu.sync_copy(hbuf, pooled_s.at[btv], add=True)
    return carry
  lax.fori_loop(0, _split(NFULL, NS, s), pbody, 0)

  @pl.when(s == NS - 1)
  def _():
    base = NFULL * CH
    build_eidx(base, NTAIL)
    pltpu.sync_copy(acc_s.at[pl.ds(base, NTAIL)], abuf.at[pl.ds(0, NTAIL)])
    cpy = pltpu.async_copy(y2_h.at[eidx.at[pl.ds(0, NTAIL)]],
                           ybuf.at[pl.ds(0, NTAIL)], semy)
    pltpu.sync_copy(deg_h.at[pl.ds(base, NTAIL)], degv.at[pl.ds(0, NTAIL)])
    pltpu.sync_copy(batch_h.at[pl.ds(base, NTAIL)], bt32)
    prep_dinv(NTAIL)
    cpy.wait()
    hcompute(NTAIL)
    pltpu.sync_copy(hbuf.at[pl.ds(0, NTAIL)], pooled_s.at[bt32], add=True)

  plsc.subcore_barrier()

  @pl.when(s == 0)
  def _():
    pltpu.sync_copy(pooled_s, out_h.at[c])


_sc_edge_pool = pl.kernel(
    _sc_edge_pool_body,
    out_type=jax.ShapeDtypeStruct((2, G, L), f32),
    mesh=plsc.VectorSubcoreMesh(core_axis_name="c", subcore_axis_name="s"),
    compiler_params=pltpu.CompilerParams(use_tc_tiling_on_sc=False,
                                         needs_layout_passes=False),
    scratch_types=[
        pltpu.VMEM((EB, CH), i32),
        pltpu.VMEM((EB, CH), i32),
        pltpu.VMEM((EB, CH), i32),
        pltpu.VMEM((CH,), i32),
    ] + [pltpu.VMEM((CH, L), f32)] * _NRING + [
        pltpu.VMEM((CH, L), f32),
        pltpu.VMEM((CH, L), f32),
        pltpu.VMEM((CH, L), f32),
        pltpu.VMEM((CH,), i32),
        pltpu.VMEM((NTAIL,), i32),
        pltpu.VMEM((L,), f32),
        pltpu.VMEM((CH,), f32),
        pltpu.VMEM((CH,), f32),
        pltpu.VMEM_SHARED((N, L), f32),
        pltpu.VMEM_SHARED((G, L), f32),
        pltpu.SemaphoreType.DMA,
        pltpu.SemaphoreType.DMA,
        pltpu.SemaphoreType.DMA,
        pltpu.SemaphoreType.DMA,
        pltpu.SemaphoreType.DMA,
    ],
)


def _tc_head_body(p_ref, cnt_ref, wfc_ref, bfc_ref, out_ref):
  ps = jnp.concatenate([p_ref[0], p_ref[1]], axis=1)
  cnt = jnp.maximum(cnt_ref[0] + cnt_ref[1], 1.0)
  pooled = ps / cnt
  logits = jnp.dot(pooled, wfc_ref[...], preferred_element_type=f32)
  logits = logits + bfc_ref[...]
  m = jnp.max(logits, axis=1, keepdims=True)
  e = jnp.exp(logits - m)
  lse = jnp.log(jnp.sum(e, axis=1, keepdims=True)) + m
  out_ref[...] = logits - lse


def _tc_head(pooled, cnt3, wfc, bfc2):
  return pl.pallas_call(
      _tc_head_body,
      out_shape=jax.ShapeDtypeStruct((G, T), f32),
  )(pooled, cnt3, wfc, bfc2)


def kernel(act, location, edge_index, batch, emb_act, emb_loc,
           W_gcn, b_gcn, W_fc, b_fc):
  act = act.astype(i32)
  location = location.astype(i32)
  edge_index = edge_index.astype(i32)
  batch = batch.astype(i32)
  ei3 = edge_index.reshape(2, ECH, CH)
  wbd = jnp.kron(jnp.eye(4, dtype=f32), W_gcn)

  x2, deg, cnt = _sc_embed_deg(act, location, ei3, batch,
                               emb_act.reshape(VA * H),
                               emb_loc.reshape(VL * H))
  y2 = _tc_matmul(x2, wbd)
  pooled = _sc_edge_pool(y2.reshape(2 * N, L), deg, ei3, batch,
                         b_gcn.reshape(2, L))
  return _tc_head(pooled, cnt.reshape(2, G, 1), W_fc, b_fc.reshape(1, T))


# speedup vs baseline: 49.6032x; 1.0976x over previous
import jax
import jax.numpy as jnp
from jax import lax
from jax.experimental import pallas as pl
from jax.experimental.pallas import tpu as pltpu
from jax.experimental.pallas import tpu_sc as plsc

N = 100000
E = 1600000
H = 32
G = 128
T = 10
VA = 1000
VL = 100

L = 16
CH = 128
NS = 16
NW = 32
NFULL = N // CH
NTAIL = N - NFULL * CH
ECH = E // CH
EB = 10
NBLK = ECH // EB
XROWS = N * H // 128

f32 = jnp.float32
i32 = jnp.int32


def _split(total, nworkers, w):
  q, r = total // nworkers, total % nworkers
  return jnp.where(w < r, q + 1, q)


def _rsqrt_nr(d):
  u = plsc.bitcast(d, i32)
  u = jnp.int32(0x5F3759DF) - lax.shift_right_logical(u, 1)
  t = plsc.bitcast(u, f32)
  for _ in range(3):
    t = t * (1.5 - 0.5 * d * t * t)
  return t


def _sc_embed_deg_body(act_h, loc_h, ei3_h, batch_h, ea_h, el_h,
                       x2_h, deg_h, cnt_h,
                       aidx, lidx, bidx, a32, l32, b32, degv,
                       xbuf2, onesv, ones32, zbufv, eav, elv,
                       deg_s, cnt_s, semsc):
  c = lax.axis_index("c")
  s = lax.axis_index("s")
  w = c * NS + s

  for i in range(CH // L):
    onesv[pl.ds(L * i, L)] = jnp.ones((L,), f32)
    zbufv[pl.ds(L * i, L)] = jnp.zeros((L,), f32)
  for i in range(NTAIL // L):
    ones32[pl.ds(L * i, L)] = jnp.ones((L,), f32)

  iotav = lax.iota(i32, L)

  def zbody(k, carry):
    base = (s + NS * k) * CH
    pltpu.sync_copy(zbufv, deg_s.at[pl.ds(base, CH)])
    return carry
  lax.fori_loop(0, _split(NFULL, NS, s), zbody, 0)

  @pl.when(s == NS - 1)
  def _():
    pltpu.sync_copy(zbufv.at[pl.ds(0, NTAIL)],
                    deg_s.at[pl.ds(NFULL * CH, NTAIL)])

  @pl.when(s == 0)
  def _():
    pltpu.sync_copy(zbufv, cnt_s)

  pltpu.sync_copy(ea_h, eav)
  pltpu.sync_copy(el_h, elv)

  plsc.subcore_barrier()

  def dbody(bk, carry):
    blk = (s + NS * bk) * EB
    pltpu.sync_copy(ei3_h.at[1, pl.ds(blk, EB)], bidx)
    scps = [pltpu.async_copy(onesv, deg_s.at[bidx.at[j]], semsc, add=True)
            for j in range(EB)]
    for cp in scps:
      cp.wait()
    return carry
  lax.fori_loop(0, _split(NBLK, NS, s), dbody, 0)

  plsc.subcore_barrier()

  def echunk(aref, lref, ngroups):
    for g in range(ngroups):
      acts = aref[pl.ds(L * g, L)] * H
      locs = lref[pl.ds(L * g, L)] * H
      lofs = (iotav + (L * g)) * H
      dv = _rsqrt_nr(degv[pl.ds(L * g, L)] + 1.0)

      def f4body(f4, carry2):
        for u in range(4):
          f = f4 * 4 + u
          fsp = jnp.full((L,), f, i32)
          va = plsc.load_gather(eav, [acts + fsp])
          vl = plsc.load_gather(elv, [locs + fsp])
          v = jnp.maximum(va + vl, 0.0) * dv
          flat = lofs + fsp
          plsc.store_scatter(
              xbuf2,
              [lax.shift_right_logical(flat, 7),
               jnp.bitwise_and(flat, 127)], v)
        return carry2
      lax.fori_loop(0, H // 4, f4body, 0)

  bidx0 = bidx.at[0]

  def ebody(k, carry):
    base = (w + NW * k) * CH
    pltpu.sync_copy(act_h.at[pl.ds(base, CH)], aidx)
    pltpu.sync_copy(loc_h.at[pl.ds(base, CH)], lidx)
    pltpu.sync_copy(batch_h.at[pl.ds(base, CH)], bidx0)
    pltpu.sync_copy(deg_s.at[pl.ds(base, CH)], degv)
    echunk(aidx, lidx, CH // L)
    pltpu.sync_copy(xbuf2, x2_h.at[pl.ds(base // 4, CH * H // 128)])
    pltpu.sync_copy(onesv, cnt_s.at[bidx0], add=True)
    return carry
  lax.fori_loop(0, _split(NFULL, NW, w), ebody, 0)

  @pl.when(w == NW - 1)
  def _():
    base = NFULL * CH
    pltpu.sync_copy(act_h.at[pl.ds(base, NTAIL)], a32)
    pltpu.sync_copy(loc_h.at[pl.ds(base, NTAIL)], l32)
    pltpu.sync_copy(batch_h.at[pl.ds(base, NTAIL)], b32)
    pltpu.sync_copy(deg_s.at[pl.ds(base, NTAIL)], degv.at[pl.ds(0, NTAIL)])
    echunk(a32, l32, NTAIL // L)
    pltpu.sync_copy(xbuf2.at[pl.ds(0, NTAIL * H // 128)],
                    x2_h.at[pl.ds(base // 4, NTAIL * H // 128)])
    pltpu.sync_copy(ones32, cnt_s.at[b32], add=True)

  plsc.subcore_barrier()

  WB = 6256

  @pl.when(c == 0)
  def _():
    @pl.when(s < NS - 1)
    def _():
      base = s * WB
      pltpu.sync_copy(deg_s.at[pl.ds(base, WB)], deg_h.at[pl.ds(base, WB)])

    @pl.when(s == NS - 1)
    def _():
      base = (NS - 1) * WB
      rem = N - base
      pltpu.sync_copy(deg_s.at[pl.ds(base, rem)], deg_h.at[pl.ds(base, rem)])

  @pl.when(s == 0)
  def _():
    pltpu.sync_copy(cnt_s, cnt_h.at[c])


_sc_embed_deg = pl.kernel(
    _sc_embed_deg_body,
    out_type=(
        jax.ShapeDtypeStruct((XROWS, 128), f32),
        jax.ShapeDtypeStruct((N,), f32),
        jax.ShapeDtypeStruct((2, G), f32),
    ),
    mesh=plsc.VectorSubcoreMesh(core_axis_name="c", subcore_axis_name="s"),
    compiler_params=pltpu.CompilerParams(use_tc_tiling_on_sc=False,
                                         needs_layout_passes=False),
    scratch_types=[
        pltpu.VMEM((CH,), i32),
        pltpu.VMEM((CH,), i32),
        pltpu.VMEM((EB, CH), i32),
        pltpu.VMEM((NTAIL,), i32),
        pltpu.VMEM((NTAIL,), i32),
        pltpu.VMEM((NTAIL,), i32),
        pltpu.VMEM((CH,), f32),
        pltpu.VMEM((CH * H // 128, 128), f32),
        pltpu.VMEM((CH,), f32),
        pltpu.VMEM((NTAIL,), f32),
        pltpu.VMEM((CH,), f32),
        pltpu.VMEM((VA * H,), f32),
        pltpu.VMEM((VL * H,), f32),
        pltpu.VMEM_SHARED((N,), f32),
        pltpu.VMEM_SHARED((G,), f32),
        pltpu.SemaphoreType.DMA,
    ],
)


_BX = 5000


def _tc_matmul_body(x_ref, w_ref, y_ref):
  y_ref[...] = jnp.dot(x_ref[...], w_ref[...], preferred_element_type=f32)


def _tc_matmul(x2, wbd):
  return pl.pallas_call(
      _tc_matmul_body,
      grid=(XROWS // _BX,),
      in_specs=[
          pl.BlockSpec((_BX, 128), lambda i: (i, 0)),
          pl.BlockSpec((128, 128), lambda i: (0, 0)),
      ],
      out_specs=pl.BlockSpec((_BX, 128), lambda i: (i, 0)),
      out_shape=jax.ShapeDtypeStruct((XROWS, 128), f32),
  )(x2, wbd)


_NRING = 5


def _sc_edge_pool_body(y2_h, deg_h, ei3_h, batch_h, b2_h,
                       out_h,
                       ridxA, cidxA, gidxA, ridxB, cidxB, gidxB,
                       eidxA, eidxB,
                       g0, g1, g2, g3, g4,
                       abufA, ybufA, degvA, btvA,
                       abufB, ybufB, degvB, btvB,
                       hbuf, bt32, bvv, dinvv,
                       acc_s, pooled_s,
                       semg, semsc, semiA, semiB,
                       semaA, semyA, semdA, sembA,
                       semaB, semyB, semdB, sembB):
  c = lax.axis_index("c")
  s = lax.axis_index("s")
  gbufs = (g0, g1, g2, g3, g4)
  iotav = lax.iota(i32, L)

  def zb_body(i, carry):
    hbuf[i, :] = jnp.zeros((L,), f32)
    return carry
  lax.fori_loop(0, CH, zb_body, 0)

  def zacc(k, carry):
    base = (s + NS * k) * CH
    pltpu.sync_copy(hbuf, acc_s.at[pl.ds(base, CH)])
    return carry
  lax.fori_loop(0, _split(NFULL, NS, s), zacc, 0)

  @pl.when(s == NS - 1)
  def _():
    pltpu.sync_copy(hbuf.at[pl.ds(0, NTAIL)],
                    acc_s.at[pl.ds(NFULL * CH, NTAIL)])

  @pl.when(s == 0)
  def _():
    pltpu.sync_copy(hbuf, pooled_s)

  plsc.subcore_barrier()

  nb = _split(NBLK, NS, s)

  def fire_idx(bk, rdst, cdst, sem):
    blk = (s + NS * bk) * EB
    cp0 = pltpu.async_copy(ei3_h.at[0, pl.ds(blk, EB)], rdst, sem)
    cp1 = pltpu.async_copy(ei3_h.at[1, pl.ds(blk, EB)], cdst, sem)
    return cp0, cp1

  def wait_idx(rdst, cdst, sem):
    pltpu.make_async_copy(ei3_h.at[0, pl.ds(0, EB)], rdst, sem).wait()
    pltpu.make_async_copy(ei3_h.at[1, pl.ds(0, EB)], cdst, sem).wait()

  def process_block(ridx2, cidx2, gidx2):
    for j in range(EB):
      for t in range(CH // L):
        v = ridx2[j, pl.ds(L * t, L)]
        gidx2[j, pl.ds(L * t, L)] = v + v + c
    gcps = [None] * EB
    for j in range(_NRING):
      gcps[j] = pltpu.async_copy(y2_h.at[gidx2.at[j]], gbufs[j], semg)
    scps = [None] * EB
    for j in range(EB):
      gcps[j].wait()
      scps[j] = pltpu.async_copy(gbufs[j % _NRING], acc_s.at[cidx2.at[j]],
                                 semsc, add=True)
      if j + _NRING < EB:
        scps[j].wait()
        gcps[j + _NRING] = pltpu.async_copy(y2_h.at[gidx2.at[j + _NRING]],
                                            gbufs[j % _NRING], semg)
    for j in range(EB - _NRING, EB):
      scps[j].wait()

  fire_idx(0, ridxA, cidxA, semiA)

  def epair(bp, carry):
    bk1 = 2 * bp + 1
    fire_idx(bk1, ridxB, cidxB, semiB)
    wait_idx(ridxA, cidxA, semiA)
    process_block(ridxA, cidxA, gidxA)
    fire_idx(jnp.minimum(2 * bp + 2, nb - 1), ridxA, cidxA, semiA)
    wait_idx(ridxB, cidxB, semiB)
    process_block(ridxB, cidxB, gidxB)
    return carry
  lax.fori_loop(0, nb // 2, epair, 0)

  wait_idx(ridxA, cidxA, semiA)

  @pl.when(s < NBLK % NS)
  def _():
    process_block(ridxA, cidxA, gidxA)

  plsc.subcore_barrier()

  pltpu.sync_copy(b2_h.at[c], bvv)
  bv = bvv[...]
  nch = _split(NFULL, NS, s)

  def build_eidx(eidx, base, nrows):
    for t in range(nrows // L):
      v = base + iotav + (L * t)
      eidx[pl.ds(L * t, L)] = v + v + c

  def fire_chunk(k, eidx, abuf, ybuf, degv, btv, sems):
    base = (s + NS * k) * CH
    build_eidx(eidx, base, CH)
    pltpu.async_copy(acc_s.at[pl.ds(base, CH)], abuf, sems[0])
    pltpu.async_copy(y2_h.at[eidx], ybuf, sems[1])
    pltpu.async_copy(deg_h.at[pl.ds(base, CH)], degv, sems[2])
    pltpu.async_copy(batch_h.at[pl.ds(base, CH)], btv, sems[3])

  def wait_chunk(abuf, ybuf, degv, btv, sems):
    pltpu.make_async_copy(y2_h.at[pl.ds(0, CH)], abuf, sems[0]).wait()
    pltpu.make_async_copy(y2_h.at[pl.ds(0, CH)], ybuf, sems[1]).wait()
    pltpu.make_async_copy(deg_h.at[pl.ds(0, CH)], degv, sems[2]).wait()
    pltpu.make_async_copy(batch_h.at[pl.ds(0, CH)], btv, sems[3]).wait()

  def compute_chunk(abuf, ybuf, degv, btv, nrows):
    for t in range(nrows // L):
      d = degv[pl.ds(L * t, L)] + 1.0
      dinvv[pl.ds(L * t, L)] = _rsqrt_nr(d)

    def hrow(i, carry2):
      for u in range(4):
        ii = i * 4 + u
        dsp = plsc.load_gather(dinvv, [jnp.full((L,), ii, i32)])
        v = (abuf[ii, :] + ybuf[ii, :]) * dsp + bv
        hbuf[ii, :] = jnp.maximum(v, 0.0)
      return carry2
    lax.fori_loop(0, nrows // 4, hrow, 0)
    pltpu.sync_copy(hbuf.at[pl.ds(0, nrows)], pooled_s.at[btv], add=True)

  semsA = (semaA, semyA, semdA, sembA)
  semsB = (semaB, semyB, semdB, sembB)
  fire_chunk(0, eidxA, abufA, ybufA, degvA, btvA, semsA)

  def ppair(kp, carry):
    fire_chunk(2 * kp + 1, eidxB, abufB, ybufB, degvB, btvB, semsB)
    wait_chunk(abufA, ybufA, degvA, btvA, semsA)
    compute_chunk(abufA, ybufA, degvA, btvA, CH)
    fire_chunk(jnp.minimum(2 * kp + 2, nch - 1), eidxA, abufA, ybufA,
               degvA, btvA, semsA)
    wait_chunk(abufB, ybufB, degvB, btvB, semsB)
    compute_chunk(abufB, ybufB, degvB, btvB, CH)
    return carry
  lax.fori_loop(0, nch // 2, ppair, 0)

  wait_chunk(abufA, ybufA, degvA, btvA, semsA)

  @pl.when(s < NFULL % NS)
  def _():
    compute_chunk(abufA, ybufA, degvA, btvA, CH)

  @pl.when(s == NS - 1)
  def _():
    base = NFULL * CH
    build_eidx(eidxA, base, NTAIL)
    pltpu.sync_copy(acc_s.at[pl.ds(base, NTAIL)], abufA.at[pl.ds(0, NTAIL)])
    cpy = pltpu.async_copy(y2_h.at[eidxA.at[pl.ds(0, NTAIL)]],
                           ybufA.at[pl.ds(0, NTAIL)], semyA)
    pltpu.sync_copy(deg_h.at[pl.ds(base, NTAIL)], degvA.at[pl.ds(0, NTAIL)])
    pltpu.sync_copy(batch_h.at[pl.ds(base, NTAIL)], bt32)
    cpy.wait()
    for t in range(NTAIL // L):
      d = degvA[pl.ds(L * t, L)] + 1.0
      dinvv[pl.ds(L * t, L)] = _rsqrt_nr(d)

    def hrow32(i, carry2):
      for u in range(4):
        ii = i * 4 + u
        dsp = plsc.load_gather(dinvv, [jnp.full((L,), ii, i32)])
        v = (abufA[ii, :] + ybufA[ii, :]) * dsp + bv
        hbuf[ii, :] = jnp.maximum(v, 0.0)
      return carry2
    lax.fori_loop(0, NTAIL // 4, hrow32, 0)
    pltpu.sync_copy(hbuf.at[pl.ds(0, NTAIL)], pooled_s.at[bt32], add=True)

  plsc.subcore_barrier()

  @pl.when(s == 0)
  def _():
    pltpu.sync_copy(pooled_s, out_h.at[c])


_sc_edge_pool = pl.kernel(
    _sc_edge_pool_body,
    out_type=jax.ShapeDtypeStruct((2, G, L), f32),
    mesh=plsc.VectorSubcoreMesh(core_axis_name="c", subcore_axis_name="s"),
    compiler_params=pltpu.CompilerParams(use_tc_tiling_on_sc=False,
                                         needs_layout_passes=False),
    scratch_types=[
        pltpu.VMEM((EB, CH), i32),
        pltpu.VMEM((EB, CH), i32),
        pltpu.VMEM((EB, CH), i32),
        pltpu.VMEM((EB, CH), i32),
        pltpu.VMEM((EB, CH), i32),
        pltpu.VMEM((EB, CH), i32),
        pltpu.VMEM((CH,), i32),
        pltpu.VMEM((CH,), i32),
    ] + [pltpu.VMEM((CH, L), f32)] * _NRING + [
        pltpu.VMEM((CH, L), f32),
        pltpu.VMEM((CH, L), f32),
        pltpu.VMEM((CH,), f32),
        pltpu.VMEM((CH,), i32),
        pltpu.VMEM((CH, L), f32),
        pltpu.VMEM((CH, L), f32),
        pltpu.VMEM((CH,), f32),
        pltpu.VMEM((CH,), i32),
        pltpu.VMEM((CH, L), f32),
        pltpu.VMEM((NTAIL,), i32),
        pltpu.VMEM((L,), f32),
        pltpu.VMEM((CH,), f32),
        pltpu.VMEM_SHARED((N, L), f32),
        pltpu.VMEM_SHARED((G, L), f32),
    ] + [pltpu.SemaphoreType.DMA] * 12,
)


def _tc_head_body(p_ref, cnt_ref, wfc_ref, bfc_ref, out_ref):
  ps = jnp.concatenate([p_ref[0], p_ref[1]], axis=1)
  cnt = jnp.maximum(cnt_ref[0] + cnt_ref[1], 1.0)
  pooled = ps / cnt
  logits = jnp.dot(pooled, wfc_ref[...], preferred_element_type=f32)
  logits = logits + bfc_ref[...]
  m = jnp.max(logits, axis=1, keepdims=True)
  e = jnp.exp(logits - m)
  lse = jnp.log(jnp.sum(e, axis=1, keepdims=True)) + m
  out_ref[...] = logits - lse


def _tc_head(pooled, cnt3, wfc, bfc2):
  return pl.pallas_call(
      _tc_head_body,
      out_shape=jax.ShapeDtypeStruct((G, T), f32),
  )(pooled, cnt3, wfc, bfc2)


def kernel(act, location, edge_index, batch, emb_act, emb_loc,
           W_gcn, b_gcn, W_fc, b_fc):
  act = act.astype(i32)
  location = location.astype(i32)
  edge_index = edge_index.astype(i32)
  batch = batch.astype(i32)
  ei3 = edge_index.reshape(2, ECH, CH)
  wbd = jnp.kron(jnp.eye(4, dtype=f32), W_gcn)

  x2, deg, cnt = _sc_embed_deg(act, location, ei3, batch,
                               emb_act.reshape(VA * H),
                               emb_loc.reshape(VL * H))
  y2 = _tc_matmul(x2, wbd)
  pooled = _sc_edge_pool(y2.reshape(2 * N, L), deg, ei3, batch,
                         b_gcn.reshape(2, L))
  return _tc_head(pooled, cnt.reshape(2, G, 1), W_fc, b_fc.reshape(1, T))


# speedup vs baseline: 54.0609x; 1.0899x over previous
import jax
import jax.numpy as jnp
from jax import lax
from jax.experimental import pallas as pl
from jax.experimental.pallas import tpu as pltpu
from jax.experimental.pallas import tpu_sc as plsc

N = 100000
E = 1600000
H = 32
G = 128
T = 10
VA = 1000
VL = 100

L = 16
CH = 128
NS = 16
NW = 32
NFULL = N // CH
NTAIL = N - NFULL * CH
ECH = E // CH
EB = 10
NBLK = ECH // EB
XROWS = N * H // 128

f32 = jnp.float32
i32 = jnp.int32


def _split(total, nworkers, w):
  q, r = total // nworkers, total % nworkers
  return jnp.where(w < r, q + 1, q)


def _rsqrt_nr(d):
  u = plsc.bitcast(d, i32)
  u = jnp.int32(0x5F3759DF) - lax.shift_right_logical(u, 1)
  t = plsc.bitcast(u, f32)
  for _ in range(3):
    t = t * (1.5 - 0.5 * d * t * t)
  return t


def _sc_embed_deg_body(act_h, loc_h, ei3_h, batch_h, ea_h, el_h,
                       x2_h, deg_h, cnt_h,
                       aidx, lidx, bidxA, bidxB, bat, a32, l32, b32, degv,
                       xbuf2, onesv, ones32, zbufv, eav, elv,
                       deg_s, cnt_s, semsc, semiA, semiB,
                       sem0, sem1, sem2, sem3):
  c = lax.axis_index("c")
  s = lax.axis_index("s")
  w = c * NS + s

  for i in range(CH // L):
    onesv[pl.ds(L * i, L)] = jnp.ones((L,), f32)
    zbufv[pl.ds(L * i, L)] = jnp.zeros((L,), f32)
  for i in range(NTAIL // L):
    ones32[pl.ds(L * i, L)] = jnp.ones((L,), f32)

  iotav = lax.iota(i32, L)

  def zbody(k, carry):
    base = (s + NS * k) * CH
    pltpu.sync_copy(zbufv, deg_s.at[pl.ds(base, CH)])
    return carry
  lax.fori_loop(0, _split(NFULL, NS, s), zbody, 0)

  @pl.when(s == NS - 1)
  def _():
    pltpu.sync_copy(zbufv.at[pl.ds(0, NTAIL)],
                    deg_s.at[pl.ds(NFULL * CH, NTAIL)])

  @pl.when(s == 0)
  def _():
    pltpu.sync_copy(zbufv, cnt_s)

  pltpu.sync_copy(ea_h, eav)
  pltpu.sync_copy(el_h, elv)

  plsc.subcore_barrier()

  nb = _split(NBLK, NS, s)

  def fire_cols(bk, dst, sem):
    blk = (s + NS * bk) * EB
    pltpu.async_copy(ei3_h.at[1, pl.ds(blk, EB)], dst, sem)

  def wait_cols(dst, sem):
    pltpu.make_async_copy(ei3_h.at[1, pl.ds(0, EB)], dst, sem).wait()

  def scatter_ones(bidx):
    scps = [pltpu.async_copy(onesv, deg_s.at[bidx.at[j]], semsc, add=True)
            for j in range(EB)]
    for cp in scps:
      cp.wait()

  fire_cols(0, bidxA, semiA)

  def dpair(bp, carry):
    fire_cols(2 * bp + 1, bidxB, semiB)
    wait_cols(bidxA, semiA)
    scatter_ones(bidxA)
    fire_cols(jnp.minimum(2 * bp + 2, nb - 1), bidxA, semiA)
    wait_cols(bidxB, semiB)
    scatter_ones(bidxB)
    return carry
  lax.fori_loop(0, nb // 2, dpair, 0)

  wait_cols(bidxA, semiA)

  @pl.when(s < NBLK % NS)
  def _():
    scatter_ones(bidxA)

  plsc.subcore_barrier()

  def echunk(aref, lref, ngroups):
    for g in range(ngroups):
      acts = aref[pl.ds(L * g, L)] * H
      locs = lref[pl.ds(L * g, L)] * H
      lofs = (iotav + (L * g)) * H
      dv = _rsqrt_nr(degv[pl.ds(L * g, L)] + 1.0)

      def f4body(f4, carry2):
        for u in range(4):
          f = f4 * 4 + u
          fsp = jnp.full((L,), f, i32)
          va = plsc.load_gather(eav, [acts + fsp])
          vl = plsc.load_gather(elv, [locs + fsp])
          v = jnp.maximum(va + vl, 0.0) * dv
          flat = lofs + fsp
          plsc.store_scatter(
              xbuf2,
              [lax.shift_right_logical(flat, 7),
               jnp.bitwise_and(flat, 127)], v)
        return carry2
      lax.fori_loop(0, H // 4, f4body, 0)

  def ebody(k, carry):
    base = (w + NW * k) * CH
    cp0 = pltpu.async_copy(act_h.at[pl.ds(base, CH)], aidx, sem0)
    cp1 = pltpu.async_copy(loc_h.at[pl.ds(base, CH)], lidx, sem1)
    cp2 = pltpu.async_copy(batch_h.at[pl.ds(base, CH)], bat, sem2)
    cp3 = pltpu.async_copy(deg_s.at[pl.ds(base, CH)], degv, sem3)
    cp0.wait()
    cp1.wait()
    cp3.wait()
    echunk(aidx, lidx, CH // L)
    pltpu.sync_copy(xbuf2, x2_h.at[pl.ds(base // 4, CH * H // 128)])
    cp2.wait()
    pltpu.sync_copy(onesv, cnt_s.at[bat], add=True)
    return carry
  lax.fori_loop(0, _split(NFULL, NW, w), ebody, 0)

  @pl.when(w == NW - 1)
  def _():
    base = NFULL * CH
    pltpu.sync_copy(act_h.at[pl.ds(base, NTAIL)], a32)
    pltpu.sync_copy(loc_h.at[pl.ds(base, NTAIL)], l32)
    pltpu.sync_copy(batch_h.at[pl.ds(base, NTAIL)], b32)
    pltpu.sync_copy(deg_s.at[pl.ds(base, NTAIL)], degv.at[pl.ds(0, NTAIL)])
    echunk(a32, l32, NTAIL // L)
    pltpu.sync_copy(xbuf2.at[pl.ds(0, NTAIL * H // 128)],
                    x2_h.at[pl.ds(base // 4, NTAIL * H // 128)])
    pltpu.sync_copy(ones32, cnt_s.at[b32], add=True)

  plsc.subcore_barrier()

  WB = 6256

  @pl.when(c == 0)
  def _():
    @pl.when(s < NS - 1)
    def _():
      base = s * WB
      pltpu.sync_copy(deg_s.at[pl.ds(base, WB)], deg_h.at[pl.ds(base, WB)])

    @pl.when(s == NS - 1)
    def _():
      base = (NS - 1) * WB
      rem = N - base
      pltpu.sync_copy(deg_s.at[pl.ds(base, rem)], deg_h.at[pl.ds(base, rem)])

  @pl.when(s == 0)
  def _():
    pltpu.sync_copy(cnt_s, cnt_h.at[c])


_sc_embed_deg = pl.kernel(
    _sc_embed_deg_body,
    out_type=(
        jax.ShapeDtypeStruct((XROWS, 128), f32),
        jax.ShapeDtypeStruct((N,), f32),
        jax.ShapeDtypeStruct((2, G), f32),
    ),
    mesh=plsc.VectorSubcoreMesh(core_axis_name="c", subcore_axis_name="s"),
    compiler_params=pltpu.CompilerParams(use_tc_tiling_on_sc=False,
                                         needs_layout_passes=False),
    scratch_types=[
        pltpu.VMEM((CH,), i32),
        pltpu.VMEM((CH,), i32),
        pltpu.VMEM((EB, CH), i32),
        pltpu.VMEM((EB, CH), i32),
        pltpu.VMEM((CH,), i32),
        pltpu.VMEM((NTAIL,), i32),
        pltpu.VMEM((NTAIL,), i32),
        pltpu.VMEM((NTAIL,), i32),
        pltpu.VMEM((CH,), f32),
        pltpu.VMEM((CH * H // 128, 128), f32),
        pltpu.VMEM((CH,), f32),
        pltpu.VMEM((NTAIL,), f32),
        pltpu.VMEM((CH,), f32),
        pltpu.VMEM((VA * H,), f32),
        pltpu.VMEM((VL * H,), f32),
        pltpu.VMEM_SHARED((N,), f32),
        pltpu.VMEM_SHARED((G,), f32),
    ] + [pltpu.SemaphoreType.DMA] * 7,
)


_BX = 5000


def _tc_matmul_body(x_ref, w_ref, y_ref):
  y_ref[...] = jnp.dot(x_ref[...], w_ref[...], preferred_element_type=f32)


def _tc_matmul(x2, wbd):
  return pl.pallas_call(
      _tc_matmul_body,
      grid=(XROWS // _BX,),
      in_specs=[
          pl.BlockSpec((_BX, 128), lambda i: (i, 0)),
          pl.BlockSpec((128, 128), lambda i: (0, 0)),
      ],
      out_specs=pl.BlockSpec((_BX, 128), lambda i: (i, 0)),
      out_shape=jax.ShapeDtypeStruct((XROWS, 128), f32),
  )(x2, wbd)


_NRING = 5


def _sc_edge_pool_body(y2_h, deg_h, ei3_h, batch_h, b2_h,
                       out_h,
                       ridxA, cidxA, gidxA, ridxB, cidxB, gidxB,
                       eidxA, eidxB,
                       g0, g1, g2, g3, g4,
                       abufA, ybufA, degvA, btvA,
                       abufB, ybufB, degvB, btvB,
                       hbuf, bt32, bvv, dinvv,
                       acc_s, pooled_s,
                       semg, semsc, semiA, semiB,
                       semaA, semyA, semdA, sembA,
                       semaB, semyB, semdB, sembB):
  c = lax.axis_index("c")
  s = lax.axis_index("s")
  gbufs = (g0, g1, g2, g3, g4)
  iotav = lax.iota(i32, L)

  def zb_body(i, carry):
    hbuf[i, :] = jnp.zeros((L,), f32)
    return carry
  lax.fori_loop(0, CH, zb_body, 0)

  def zacc(k, carry):
    base = (s + NS * k) * CH
    pltpu.sync_copy(hbuf, acc_s.at[pl.ds(base, CH)])
    return carry
  lax.fori_loop(0, _split(NFULL, NS, s), zacc, 0)

  @pl.when(s == NS - 1)
  def _():
    pltpu.sync_copy(hbuf.at[pl.ds(0, NTAIL)],
                    acc_s.at[pl.ds(NFULL * CH, NTAIL)])

  @pl.when(s == 0)
  def _():
    pltpu.sync_copy(hbuf, pooled_s)

  plsc.subcore_barrier()

  nb = _split(NBLK, NS, s)

  def fire_idx(bk, rdst, cdst, sem):
    blk = (s + NS * bk) * EB
    cp0 = pltpu.async_copy(ei3_h.at[0, pl.ds(blk, EB)], rdst, sem)
    cp1 = pltpu.async_copy(ei3_h.at[1, pl.ds(blk, EB)], cdst, sem)
    return cp0, cp1

  def wait_idx(rdst, cdst, sem):
    pltpu.make_async_copy(ei3_h.at[0, pl.ds(0, EB)], rdst, sem).wait()
    pltpu.make_async_copy(ei3_h.at[1, pl.ds(0, EB)], cdst, sem).wait()

  def process_block(ridx2, cidx2, gidx2):
    for j in range(EB):
      for t in range(CH // L):
        v = ridx2[j, pl.ds(L * t, L)]
        gidx2[j, pl.ds(L * t, L)] = v + v + c
    gcps = [None] * EB
    for j in range(_NRING):
      gcps[j] = pltpu.async_copy(y2_h.at[gidx2.at[j]], gbufs[j], semg)
    scps = [None] * EB
    for j in range(EB):
      gcps[j].wait()
      scps[j] = pltpu.async_copy(gbufs[j % _NRING], acc_s.at[cidx2.at[j]],
                                 semsc, add=True)
      if j + _NRING < EB:
        scps[j].wait()
        gcps[j + _NRING] = pltpu.async_copy(y2_h.at[gidx2.at[j + _NRING]],
                                            gbufs[j % _NRING], semg)
    for j in range(EB - _NRING, EB):
      scps[j].wait()

  fire_idx(0, ridxA, cidxA, semiA)

  def epair(bp, carry):
    bk1 = 2 * bp + 1
    fire_idx(bk1, ridxB, cidxB, semiB)
    wait_idx(ridxA, cidxA, semiA)
    process_block(ridxA, cidxA, gidxA)
    fire_idx(jnp.minimum(2 * bp + 2, nb - 1), ridxA, cidxA, semiA)
    wait_idx(ridxB, cidxB, semiB)
    process_block(ridxB, cidxB, gidxB)
    return carry
  lax.fori_loop(0, nb // 2, epair, 0)

  wait_idx(ridxA, cidxA, semiA)

  @pl.when(s < NBLK % NS)
  def _():
    process_block(ridxA, cidxA, gidxA)

  plsc.subcore_barrier()

  pltpu.sync_copy(b2_h.at[c], bvv)
  bv = bvv[...]
  nch = _split(NFULL, NS, s)

  def build_eidx(eidx, base, nrows):
    for t in range(nrows // L):
      v = base + iotav + (L * t)
      eidx[pl.ds(L * t, L)] = v + v + c

  def fire_chunk(k, eidx, abuf, ybuf, degv, btv, sems):
    base = (s + NS * k) * CH
    build_eidx(eidx, base, CH)
    pltpu.async_copy(acc_s.at[pl.ds(base, CH)], abuf, sems[0])
    pltpu.async_copy(y2_h.at[eidx], ybuf, sems[1])
    pltpu.async_copy(deg_h.at[pl.ds(base, CH)], degv, sems[2])
    pltpu.async_copy(batch_h.at[pl.ds(base, CH)], btv, sems[3])

  def wait_chunk(abuf, ybuf, degv, btv, sems):
    pltpu.make_async_copy(y2_h.at[pl.ds(0, CH)], abuf, sems[0]).wait()
    pltpu.make_async_copy(y2_h.at[pl.ds(0, CH)], ybuf, sems[1]).wait()
    pltpu.make_async_copy(deg_h.at[pl.ds(0, CH)], degv, sems[2]).wait()
    pltpu.make_async_copy(batch_h.at[pl.ds(0, CH)], btv, sems[3]).wait()

  def compute_chunk(abuf, ybuf, degv, btv, nrows):
    for t in range(nrows // L):
      d = degv[pl.ds(L * t, L)] + 1.0
      dinvv[pl.ds(L * t, L)] = _rsqrt_nr(d)

    def hrow(i, carry2):
      for u in range(4):
        ii = i * 4 + u
        dsp = plsc.load_gather(dinvv, [jnp.full((L,), ii, i32)])
        v = (abuf[ii, :] + ybuf[ii, :]) * dsp + bv
        hbuf[ii, :] = jnp.maximum(v, 0.0)
      return carry2
    lax.fori_loop(0, nrows // 4, hrow, 0)
    pltpu.sync_copy(hbuf.at[pl.ds(0, nrows)], pooled_s.at[btv], add=True)

  semsA = (semaA, semyA, semdA, sembA)
  semsB = (semaB, semyB, semdB, sembB)
  fire_chunk(0, eidxA, abufA, ybufA, degvA, btvA, semsA)

  def ppair(kp, carry):
    fire_chunk(2 * kp + 1, eidxB, abufB, ybufB, degvB, btvB, semsB)
    wait_chunk(abufA, ybufA, degvA, btvA, semsA)
    compute_chunk(abufA, ybufA, degvA, btvA, CH)
    fire_chunk(jnp.minimum(2 * kp + 2, nch - 1), eidxA, abufA, ybufA,
               degvA, btvA, semsA)
    wait_chunk(abufB, ybufB, degvB, btvB, semsB)
    compute_chunk(abufB, ybufB, degvB, btvB, CH)
    return carry
  lax.fori_loop(0, nch // 2, ppair, 0)

  wait_chunk(abufA, ybufA, degvA, btvA, semsA)

  @pl.when(s < NFULL % NS)
  def _():
    compute_chunk(abufA, ybufA, degvA, btvA, CH)

  @pl.when(s == NS - 1)
  def _():
    base = NFULL * CH
    build_eidx(eidxA, base, NTAIL)
    pltpu.sync_copy(acc_s.at[pl.ds(base, NTAIL)], abufA.at[pl.ds(0, NTAIL)])
    cpy = pltpu.async_copy(y2_h.at[eidxA.at[pl.ds(0, NTAIL)]],
                           ybufA.at[pl.ds(0, NTAIL)], semyA)
    pltpu.sync_copy(deg_h.at[pl.ds(base, NTAIL)], degvA.at[pl.ds(0, NTAIL)])
    pltpu.sync_copy(batch_h.at[pl.ds(base, NTAIL)], bt32)
    cpy.wait()
    for t in range(NTAIL // L):
      d = degvA[pl.ds(L * t, L)] + 1.0
      dinvv[pl.ds(L * t, L)] = _rsqrt_nr(d)

    def hrow32(i, carry2):
      for u in range(4):
        ii = i * 4 + u
        dsp = plsc.load_gather(dinvv, [jnp.full((L,), ii, i32)])
        v = (abufA[ii, :] + ybufA[ii, :]) * dsp + bv
        hbuf[ii, :] = jnp.maximum(v, 0.0)
      return carry2
    lax.fori_loop(0, NTAIL // 4, hrow32, 0)
    pltpu.sync_copy(hbuf.at[pl.ds(0, NTAIL)], pooled_s.at[bt32], add=True)

  plsc.subcore_barrier()

  @pl.when(s == 0)
  def _():
    pltpu.sync_copy(pooled_s, out_h.at[c])


_sc_edge_pool = pl.kernel(
    _sc_edge_pool_body,
    out_type=jax.ShapeDtypeStruct((2, G, L), f32),
    mesh=plsc.VectorSubcoreMesh(core_axis_name="c", subcore_axis_name="s"),
    compiler_params=pltpu.CompilerParams(use_tc_tiling_on_sc=False,
                                         needs_layout_passes=False),
    scratch_types=[
        pltpu.VMEM((EB, CH), i32),
        pltpu.VMEM((EB, CH), i32),
        pltpu.VMEM((EB, CH), i32),
        pltpu.VMEM((EB, CH), i32),
        pltpu.VMEM((EB, CH), i32),
        pltpu.VMEM((EB, CH), i32),
        pltpu.VMEM((CH,), i32),
        pltpu.VMEM((CH,), i32),
    ] + [pltpu.VMEM((CH, L), f32)] * _NRING + [
        pltpu.VMEM((CH, L), f32),
        pltpu.VMEM((CH, L), f32),
        pltpu.VMEM((CH,), f32),
        pltpu.VMEM((CH,), i32),
        pltpu.VMEM((CH, L), f32),
        pltpu.VMEM((CH, L), f32),
        pltpu.VMEM((CH,), f32),
        pltpu.VMEM((CH,), i32),
        pltpu.VMEM((CH, L), f32),
        pltpu.VMEM((NTAIL,), i32),
        pltpu.VMEM((L,), f32),
        pltpu.VMEM((CH,), f32),
        pltpu.VMEM_SHARED((N, L), f32),
        pltpu.VMEM_SHARED((G, L), f32),
    ] + [pltpu.SemaphoreType.DMA] * 12,
)


def _tc_head_body(p_ref, cnt_ref, wfc_ref, bfc_ref, out_ref):
  ps = jnp.concatenate([p_ref[0], p_ref[1]], axis=1)
  cnt = jnp.maximum(cnt_ref[0] + cnt_ref[1], 1.0)
  pooled = ps / cnt
  logits = jnp.dot(pooled, wfc_ref[...], preferred_element_type=f32)
  logits = logits + bfc_ref[...]
  m = jnp.max(logits, axis=1, keepdims=True)
  e = jnp.exp(logits - m)
  lse = jnp.log(jnp.sum(e, axis=1, keepdims=True)) + m
  out_ref[...] = logits - lse


def _tc_head(pooled, cnt3, wfc, bfc2):
  return pl.pallas_call(
      _tc_head_body,
      out_shape=jax.ShapeDtypeStruct((G, T), f32),
  )(pooled, cnt3, wfc, bfc2)


def kernel(act, location, edge_index, batch, emb_act, emb_loc,
           W_gcn, b_gcn, W_fc, b_fc):
  act = act.astype(i32)
  location = location.astype(i32)
  edge_index = edge_index.astype(i32)
  batch = batch.astype(i32)
  ei3 = edge_index.reshape(2, ECH, CH)
  wbd = jnp.kron(jnp.eye(4, dtype=f32), W_gcn)

  x2, deg, cnt = _sc_embed_deg(act, location, ei3, batch,
                               emb_act.reshape(VA * H),
                               emb_loc.reshape(VL * H))
  y2 = _tc_matmul(x2, wbd)
  pooled = _sc_edge_pool(y2.reshape(2 * N, L), deg, ei3, batch,
                         b_gcn.reshape(2, L))
  return _tc_head(pooled, cnt.reshape(2, G, 1), W_fc, b_fc.reshape(1, T))


# speedup vs baseline: 54.2200x; 1.0029x over previous
import jax
import jax.numpy as jnp
from jax import lax
from jax.experimental import pallas as pl
from jax.experimental.pallas import tpu as pltpu
from jax.experimental.pallas import tpu_sc as plsc

N = 100000
E = 1600000
H = 32
G = 128
T = 10
VA = 1000
VL = 100

L = 16
CH = 128
NS = 16
NW = 32
NFULL = N // CH
NTAIL = N - NFULL * CH
ECH = E // CH
EB = 10
NBLK = ECH // EB
XROWS = N * H // 128

f32 = jnp.float32
i32 = jnp.int32


def _split(total, nworkers, w):
  q, r = total // nworkers, total % nworkers
  return jnp.where(w < r, q + 1, q)


def _rsqrt_nr(d):
  u = plsc.bitcast(d, i32)
  u = jnp.int32(0x5F3759DF) - lax.shift_right_logical(u, 1)
  t = plsc.bitcast(u, f32)
  for _ in range(3):
    t = t * (1.5 - 0.5 * d * t * t)
  return t


def _sc_embed_deg_body(act_h, loc_h, ei3_h, batch_h, ea_h, el_h,
                       x2_h, deg_h, cnt_h,
                       aidx, lidx, bidxA, bidxB, bat, a32, l32, b32, degv,
                       xbuf2, onesv, ones32, zbufv, eav, elv,
                       deg_s, cnt_s, semsc, semiA, semiB,
                       sem0, sem1, sem2, sem3):
  c = lax.axis_index("c")
  s = lax.axis_index("s")
  w = c * NS + s

  for i in range(CH // L):
    onesv[pl.ds(L * i, L)] = jnp.ones((L,), f32)
    zbufv[pl.ds(L * i, L)] = jnp.zeros((L,), f32)
  for i in range(NTAIL // L):
    ones32[pl.ds(L * i, L)] = jnp.ones((L,), f32)

  iotav = lax.iota(i32, L)

  def zbody(k, carry):
    base = (s + NS * k) * CH
    pltpu.sync_copy(zbufv, deg_s.at[pl.ds(base, CH)])
    return carry
  lax.fori_loop(0, _split(NFULL, NS, s), zbody, 0)

  @pl.when(s == NS - 1)
  def _():
    pltpu.sync_copy(zbufv.at[pl.ds(0, NTAIL)],
                    deg_s.at[pl.ds(NFULL * CH, NTAIL)])

  @pl.when(s == 0)
  def _():
    pltpu.sync_copy(zbufv, cnt_s)

  pltpu.sync_copy(ea_h, eav)
  pltpu.sync_copy(el_h, elv)

  plsc.subcore_barrier()

  nb = _split(NBLK, NS, s)

  def fire_cols(bk, dst, sem):
    blk = (s + NS * bk) * EB
    pltpu.async_copy(ei3_h.at[1, pl.ds(blk, EB)], dst, sem)

  def wait_cols(dst, sem):
    pltpu.make_async_copy(ei3_h.at[1, pl.ds(0, EB)], dst, sem).wait()

  def scatter_ones(bidx):
    scps = [pltpu.async_copy(onesv, deg_s.at[bidx.at[j]], semsc, add=True)
            for j in range(EB)]
    for cp in scps:
      cp.wait()

  fire_cols(0, bidxA, semiA)

  def dpair(bp, carry):
    fire_cols(2 * bp + 1, bidxB, semiB)
    wait_cols(bidxA, semiA)
    scatter_ones(bidxA)
    fire_cols(jnp.minimum(2 * bp + 2, nb - 1), bidxA, semiA)
    wait_cols(bidxB, semiB)
    scatter_ones(bidxB)
    return carry
  lax.fori_loop(0, nb // 2, dpair, 0)

  wait_cols(bidxA, semiA)

  @pl.when(nb % 2 == 1)
  def _():
    scatter_ones(bidxA)

  plsc.subcore_barrier()

  def echunk(aref, lref, ngroups):
    for g in range(ngroups):
      acts = aref[pl.ds(L * g, L)] * H
      locs = lref[pl.ds(L * g, L)] * H
      lofs = (iotav + (L * g)) * H
      dv = _rsqrt_nr(degv[pl.ds(L * g, L)] + 1.0)

      def f4body(f4, carry2):
        for u in range(4):
          f = f4 * 4 + u
          fsp = jnp.full((L,), f, i32)
          va = plsc.load_gather(eav, [acts + fsp])
          vl = plsc.load_gather(elv, [locs + fsp])
          v = jnp.maximum(va + vl, 0.0) * dv
          flat = lofs + fsp
          plsc.store_scatter(
              xbuf2,
              [lax.shift_right_logical(flat, 7),
               jnp.bitwise_and(flat, 127)], v)
        return carry2
      lax.fori_loop(0, H // 4, f4body, 0)

  def ebody(k, carry):
    base = (w + NW * k) * CH
    cp0 = pltpu.async_copy(act_h.at[pl.ds(base, CH)], aidx, sem0)
    cp1 = pltpu.async_copy(loc_h.at[pl.ds(base, CH)], lidx, sem1)
    cp2 = pltpu.async_copy(batch_h.at[pl.ds(base, CH)], bat, sem2)
    cp3 = pltpu.async_copy(deg_s.at[pl.ds(base, CH)], degv, sem3)
    cp0.wait()
    cp1.wait()
    cp3.wait()
    echunk(aidx, lidx, CH // L)
    pltpu.sync_copy(xbuf2, x2_h.at[pl.ds(base // 4, CH * H // 128)])
    cp2.wait()
    pltpu.sync_copy(onesv, cnt_s.at[bat], add=True)
    return carry
  lax.fori_loop(0, _split(NFULL, NW, w), ebody, 0)

  @pl.when(w == NW - 1)
  def _():
    base = NFULL * CH
    pltpu.sync_copy(act_h.at[pl.ds(base, NTAIL)], a32)
    pltpu.sync_copy(loc_h.at[pl.ds(base, NTAIL)], l32)
    pltpu.sync_copy(batch_h.at[pl.ds(base, NTAIL)], b32)
    pltpu.sync_copy(deg_s.at[pl.ds(base, NTAIL)], degv.at[pl.ds(0, NTAIL)])
    echunk(a32, l32, NTAIL // L)
    pltpu.sync_copy(xbuf2.at[pl.ds(0, NTAIL * H // 128)],
                    x2_h.at[pl.ds(base // 4, NTAIL * H // 128)])
    pltpu.sync_copy(ones32, cnt_s.at[b32], add=True)

  plsc.subcore_barrier()

  WB = 6256

  @pl.when(c == 0)
  def _():
    @pl.when(s < NS - 1)
    def _():
      base = s * WB
      pltpu.sync_copy(deg_s.at[pl.ds(base, WB)], deg_h.at[pl.ds(base, WB)])

    @pl.when(s == NS - 1)
    def _():
      base = (NS - 1) * WB
      rem = N - base
      pltpu.sync_copy(deg_s.at[pl.ds(base, rem)], deg_h.at[pl.ds(base, rem)])

  @pl.when(s == 0)
  def _():
    pltpu.sync_copy(cnt_s, cnt_h.at[c])


_sc_embed_deg = pl.kernel(
    _sc_embed_deg_body,
    out_type=(
        jax.ShapeDtypeStruct((XROWS, 128), f32),
        jax.ShapeDtypeStruct((N,), f32),
        jax.ShapeDtypeStruct((2, G), f32),
    ),
    mesh=plsc.VectorSubcoreMesh(core_axis_name="c", subcore_axis_name="s"),
    compiler_params=pltpu.CompilerParams(use_tc_tiling_on_sc=False,
                                         needs_layout_passes=False),
    scratch_types=[
        pltpu.VMEM((CH,), i32),
        pltpu.VMEM((CH,), i32),
        pltpu.VMEM((EB, CH), i32),
        pltpu.VMEM((EB, CH), i32),
        pltpu.VMEM((CH,), i32),
        pltpu.VMEM((NTAIL,), i32),
        pltpu.VMEM((NTAIL,), i32),
        pltpu.VMEM((NTAIL,), i32),
        pltpu.VMEM((CH,), f32),
        pltpu.VMEM((CH * H // 128, 128), f32),
        pltpu.VMEM((CH,), f32),
        pltpu.VMEM((NTAIL,), f32),
        pltpu.VMEM((CH,), f32),
        pltpu.VMEM((VA * H,), f32),
        pltpu.VMEM((VL * H,), f32),
        pltpu.VMEM_SHARED((N,), f32),
        pltpu.VMEM_SHARED((G,), f32),
    ] + [pltpu.SemaphoreType.DMA] * 7,
)


_BX = 5000


def _tc_matmul_body(x_ref, w_ref, y_ref):
  y_ref[...] = jnp.dot(x_ref[...], w_ref[...], preferred_element_type=f32)


def _tc_matmul(x2, wbd):
  return pl.pallas_call(
      _tc_matmul_body,
      grid=(XROWS // _BX,),
      in_specs=[
          pl.BlockSpec((_BX, 128), lambda i: (i, 0)),
          pl.BlockSpec((128, 128), lambda i: (0, 0)),
      ],
      out_specs=pl.BlockSpec((_BX, 128), lambda i: (i, 0)),
      out_shape=jax.ShapeDtypeStruct((XROWS, 128), f32),
  )(x2, wbd)


_NRING = 5


def _sc_edge_pool_body(y2_h, deg_h, ei3_h, batch_h, b2_h,
                       out_h,
                       ridxA, cidxA, gidxA, ridxB, cidxB,
                       eidxA, eidxB,
                       g0, g1, g2, g3, g4,
                       abufA, ybufA, degvA, btvA,
                       abufB, ybufB, degvB, btvB,
                       hbuf, bt32, bvv, dinvv,
                       acc_s, pooled_s,
                       semg, semsc, semiA, semiB,
                       semaA, semyA, semdA, sembA,
                       semaB, semyB, semdB, sembB):
  c = lax.axis_index("c")
  s = lax.axis_index("s")
  gbufs = (g0, g1, g2, g3, g4)
  iotav = lax.iota(i32, L)

  def zb_body(i, carry):
    hbuf[i, :] = jnp.zeros((L,), f32)
    return carry
  lax.fori_loop(0, CH, zb_body, 0)

  def zacc(k, carry):
    base = (s + NS * k) * CH
    pltpu.sync_copy(hbuf, acc_s.at[pl.ds(base, CH)])
    return carry
  lax.fori_loop(0, _split(NFULL, NS, s), zacc, 0)

  @pl.when(s == NS - 1)
  def _():
    pltpu.sync_copy(hbuf.at[pl.ds(0, NTAIL)],
                    acc_s.at[pl.ds(NFULL * CH, NTAIL)])

  @pl.when(s == 0)
  def _():
    pltpu.sync_copy(hbuf, pooled_s)

  plsc.subcore_barrier()

  nb = _split(NBLK, NS, s)

  def fire_idx(bk, rdst, cdst, sem):
    blk = (s + NS * bk) * EB
    cp0 = pltpu.async_copy(ei3_h.at[0, pl.ds(blk, EB)], rdst, sem)
    cp1 = pltpu.async_copy(ei3_h.at[1, pl.ds(blk, EB)], cdst, sem)
    return cp0, cp1

  def wait_idx(rdst, cdst, sem):
    pltpu.make_async_copy(ei3_h.at[0, pl.ds(0, EB)], rdst, sem).wait()
    pltpu.make_async_copy(ei3_h.at[1, pl.ds(0, EB)], cdst, sem).wait()

  def process_block(ridx2, cidx2, gidx2):
    for j in range(_NRING):
      for t in range(CH // L):
        v = ridx2[j, pl.ds(L * t, L)]
        gidx2[j, pl.ds(L * t, L)] = v + v + c
    gcps = [None] * EB
    for j in range(_NRING):
      gcps[j] = pltpu.async_copy(y2_h.at[gidx2.at[j]], gbufs[j], semg)
    for j in range(_NRING, EB):
      for t in range(CH // L):
        v = ridx2[j, pl.ds(L * t, L)]
        gidx2[j, pl.ds(L * t, L)] = v + v + c
    scps = [None] * EB
    for j in range(EB):
      gcps[j].wait()
      scps[j] = pltpu.async_copy(gbufs[j % _NRING], acc_s.at[cidx2.at[j]],
                                 semsc, add=True)
      if j + _NRING < EB:
        scps[j].wait()
        gcps[j + _NRING] = pltpu.async_copy(y2_h.at[gidx2.at[j + _NRING]],
                                            gbufs[j % _NRING], semg)
    for j in range(EB - _NRING, EB):
      scps[j].wait()

  fire_idx(0, ridxA, cidxA, semiA)

  def epair(bp, carry):
    bk1 = 2 * bp + 1
    fire_idx(bk1, ridxB, cidxB, semiB)
    wait_idx(ridxA, cidxA, semiA)
    process_block(ridxA, cidxA, gidxA)
    fire_idx(jnp.minimum(2 * bp + 2, nb - 1), ridxA, cidxA, semiA)
    wait_idx(ridxB, cidxB, semiB)
    process_block(ridxB, cidxB, gidxA)
    return carry
  lax.fori_loop(0, nb // 2, epair, 0)

  wait_idx(ridxA, cidxA, semiA)

  @pl.when(nb % 2 == 1)
  def _():
    process_block(ridxA, cidxA, gidxA)

  plsc.subcore_barrier()

  pltpu.sync_copy(b2_h.at[c], bvv)
  bv = bvv[...]
  nch = _split(NFULL, NS, s)

  def build_eidx(eidx, base, nrows):
    for t in range(nrows // L):
      v = base + iotav + (L * t)
      eidx[pl.ds(L * t, L)] = v + v + c

  def fire_chunk(k, eidx, abuf, ybuf, degv, btv, sems):
    base = (s + NS * k) * CH
    build_eidx(eidx, base, CH)
    pltpu.async_copy(acc_s.at[pl.ds(base, CH)], abuf, sems[0])
    pltpu.async_copy(y2_h.at[eidx], ybuf, sems[1])
    pltpu.async_copy(deg_h.at[pl.ds(base, CH)], degv, sems[2])
    pltpu.async_copy(batch_h.at[pl.ds(base, CH)], btv, sems[3])

  def wait_chunk(abuf, ybuf, degv, btv, sems):
    pltpu.make_async_copy(y2_h.at[pl.ds(0, CH)], abuf, sems[0]).wait()
    pltpu.make_async_copy(y2_h.at[pl.ds(0, CH)], ybuf, sems[1]).wait()
    pltpu.make_async_copy(deg_h.at[pl.ds(0, CH)], degv, sems[2]).wait()
    pltpu.make_async_copy(batch_h.at[pl.ds(0, CH)], btv, sems[3]).wait()

  def compute_chunk(abuf, ybuf, degv, btv, nrows):
    for t in range(nrows // L):
      d = degv[pl.ds(L * t, L)] + 1.0
      dinvv[pl.ds(L * t, L)] = _rsqrt_nr(d)

    def hrow(i, carry2):
      for u in range(4):
        ii = i * 4 + u
        dsp = plsc.load_gather(dinvv, [jnp.full((L,), ii, i32)])
        v = (abuf[ii, :] + ybuf[ii, :]) * dsp + bv
        hbuf[ii, :] = jnp.maximum(v, 0.0)
      return carry2
    lax.fori_loop(0, nrows // 4, hrow, 0)
    pltpu.sync_copy(hbuf.at[pl.ds(0, nrows)], pooled_s.at[btv], add=True)

  semsA = (semaA, semyA, semdA, sembA)
  semsB = (semaB, semyB, semdB, sembB)
  fire_chunk(0, eidxA, abufA, ybufA, degvA, btvA, semsA)

  def ppair(kp, carry):
    fire_chunk(2 * kp + 1, eidxB, abufB, ybufB, degvB, btvB, semsB)
    wait_chunk(abufA, ybufA, degvA, btvA, semsA)
    compute_chunk(abufA, ybufA, degvA, btvA, CH)
    fire_chunk(jnp.minimum(2 * kp + 2, nch - 1), eidxA, abufA, ybufA,
               degvA, btvA, semsA)
    wait_chunk(abufB, ybufB, degvB, btvB, semsB)
    compute_chunk(abufB, ybufB, degvB, btvB, CH)
    return carry
  lax.fori_loop(0, nch // 2, ppair, 0)

  wait_chunk(abufA, ybufA, degvA, btvA, semsA)

  @pl.when(nch % 2 == 1)
  def _():
    compute_chunk(abufA, ybufA, degvA, btvA, CH)

  @pl.when(s == NS - 1)
  def _():
    base = NFULL * CH
    build_eidx(eidxA, base, NTAIL)
    pltpu.sync_copy(acc_s.at[pl.ds(base, NTAIL)], abufA.at[pl.ds(0, NTAIL)])
    cpy = pltpu.async_copy(y2_h.at[eidxA.at[pl.ds(0, NTAIL)]],
                           ybufA.at[pl.ds(0, NTAIL)], semyA)
    pltpu.sync_copy(deg_h.at[pl.ds(base, NTAIL)], degvA.at[pl.ds(0, NTAIL)])
    pltpu.sync_copy(batch_h.at[pl.ds(base, NTAIL)], bt32)
    cpy.wait()
    for t in range(NTAIL // L):
      d = degvA[pl.ds(L * t, L)] + 1.0
      dinvv[pl.ds(L * t, L)] = _rsqrt_nr(d)

    def hrow32(i, carry2):
      for u in range(4):
        ii = i * 4 + u
        dsp = plsc.load_gather(dinvv, [jnp.full((L,), ii, i32)])
        v = (abufA[ii, :] + ybufA[ii, :]) * dsp + bv
        hbuf[ii, :] = jnp.maximum(v, 0.0)
      return carry2
    lax.fori_loop(0, NTAIL // 4, hrow32, 0)
    pltpu.sync_copy(hbuf.at[pl.ds(0, NTAIL)], pooled_s.at[bt32], add=True)

  plsc.subcore_barrier()

  @pl.when(s == 0)
  def _():
    pltpu.sync_copy(pooled_s, out_h.at[c])


_sc_edge_pool = pl.kernel(
    _sc_edge_pool_body,
    out_type=jax.ShapeDtypeStruct((2, G, L), f32),
    mesh=plsc.VectorSubcoreMesh(core_axis_name="c", subcore_axis_name="s"),
    compiler_params=pltpu.CompilerParams(use_tc_tiling_on_sc=False,
                                         needs_layout_passes=False),
    scratch_types=[
        pltpu.VMEM((EB, CH), i32),
        pltpu.VMEM((EB, CH), i32),
        pltpu.VMEM((EB, CH), i32),
        pltpu.VMEM((EB, CH), i32),
        pltpu.VMEM((EB, CH), i32),
        pltpu.VMEM((CH,), i32),
        pltpu.VMEM((CH,), i32),
    ] + [pltpu.VMEM((CH, L), f32)] * _NRING + [
        pltpu.VMEM((CH, L), f32),
        pltpu.VMEM((CH, L), f32),
        pltpu.VMEM((CH,), f32),
        pltpu.VMEM((CH,), i32),
        pltpu.VMEM((CH, L), f32),
        pltpu.VMEM((CH, L), f32),
        pltpu.VMEM((CH,), f32),
        pltpu.VMEM((CH,), i32),
        pltpu.VMEM((CH, L), f32),
        pltpu.VMEM((NTAIL,), i32),
        pltpu.VMEM((L,), f32),
        pltpu.VMEM((CH,), f32),
        pltpu.VMEM_SHARED((N, L), f32),
        pltpu.VMEM_SHARED((G, L), f32),
    ] + [pltpu.SemaphoreType.DMA] * 12,
)


def _tc_head_body(p_ref, cnt_ref, wfc_ref, bfc_ref, out_ref):
  ps = jnp.concatenate([p_ref[0], p_ref[1]], axis=1)
  cnt = jnp.maximum(cnt_ref[0] + cnt_ref[1], 1.0)
  pooled = ps / cnt
  logits = jnp.dot(pooled, wfc_ref[...], preferred_element_type=f32)
  logits = logits + bfc_ref[...]
  m = jnp.max(logits, axis=1, keepdims=True)
  e = jnp.exp(logits - m)
  lse = jnp.log(jnp.sum(e, axis=1, keepdims=True)) + m
  out_ref[...] = logits - lse


def _tc_head(pooled, cnt3, wfc, bfc2):
  return pl.pallas_call(
      _tc_head_body,
      out_shape=jax.ShapeDtypeStruct((G, T), f32),
  )(pooled, cnt3, wfc, bfc2)


def kernel(act, location, edge_index, batch, emb_act, emb_loc,
           W_gcn, b_gcn, W_fc, b_fc):
  act = act.astype(i32)
  location = location.astype(i32)
  edge_index = edge_index.astype(i32)
  batch = batch.astype(i32)
  ei3 = edge_index.reshape(2, ECH, CH)
  wbd = jnp.kron(jnp.eye(4, dtype=f32), W_gcn)

  x2, deg, cnt = _sc_embed_deg(act, location, ei3, batch,
                               emb_act.reshape(VA * H),
                               emb_loc.reshape(VL * H))
  y2 = _tc_matmul(x2, wbd)
  pooled = _sc_edge_pool(y2.reshape(2 * N, L), deg, ei3, batch,
                         b_gcn.reshape(2, L))
  return _tc_head(pooled, cnt.reshape(2, G, 1), W_fc, b_fc.reshape(1, T))


# speedup vs baseline: 54.9381x; 1.0132x over previous
import jax
import jax.numpy as jnp
from jax import lax
from jax.experimental import pallas as pl
from jax.experimental.pallas import tpu as pltpu
from jax.experimental.pallas import tpu_sc as plsc

N = 100000
E = 1600000
H = 32
G = 128
T = 10
VA = 1000
VL = 100

L = 16
CH = 128
NS = 16
NW = 32
NFULL = N // CH
NTAIL = N - NFULL * CH
ECH = E // CH
EB = 10
NBLK = ECH // EB
EBD = 20
NBLKD = ECH // EBD
XROWS = N * H // 128

f32 = jnp.float32
i32 = jnp.int32


def _split(total, nworkers, w):
  q, r = total // nworkers, total % nworkers
  return jnp.where(w < r, q + 1, q)


def _rsqrt_nr(d):
  u = plsc.bitcast(d, i32)
  u = jnp.int32(0x5F3759DF) - lax.shift_right_logical(u, 1)
  t = plsc.bitcast(u, f32)
  for _ in range(3):
    t = t * (1.5 - 0.5 * d * t * t)
  return t


def _sc_embed_deg_body(act_h, loc_h, ei3_h, batch_h, ea_h, el_h,
                       x2_h, deg_h, cnt_h,
                       aidx, lidx, bidxA, bidxB, bat, a32, l32, b32, degv,
                       xbuf2, onesv, ones32, zbufv, eav, elv,
                       deg_s, cnt_s, semsc, semiA, semiB,
                       sem0, sem1, sem2, sem3):
  c = lax.axis_index("c")
  s = lax.axis_index("s")
  w = c * NS + s

  for i in range(CH // L):
    onesv[pl.ds(L * i, L)] = jnp.ones((L,), f32)
    zbufv[pl.ds(L * i, L)] = jnp.zeros((L,), f32)
  for i in range(NTAIL // L):
    ones32[pl.ds(L * i, L)] = jnp.ones((L,), f32)

  iotav = lax.iota(i32, L)

  def zbody(k, carry):
    base = (s + NS * k) * CH
    pltpu.sync_copy(zbufv, deg_s.at[pl.ds(base, CH)])
    return carry
  lax.fori_loop(0, _split(NFULL, NS, s), zbody, 0)

  @pl.when(s == NS - 1)
  def _():
    pltpu.sync_copy(zbufv.at[pl.ds(0, NTAIL)],
                    deg_s.at[pl.ds(NFULL * CH, NTAIL)])

  @pl.when(s == 0)
  def _():
    pltpu.sync_copy(zbufv, cnt_s)

  pltpu.sync_copy(ea_h, eav)
  pltpu.sync_copy(el_h, elv)

  plsc.subcore_barrier()

  nb = _split(NBLKD, NS, s)

  def fire_cols(bk, dst, sem):
    blk = (s + NS * bk) * EBD
    pltpu.async_copy(ei3_h.at[1, pl.ds(blk, EBD)], dst, sem)

  def wait_cols(dst, sem):
    pltpu.make_async_copy(ei3_h.at[1, pl.ds(0, EBD)], dst, sem).wait()

  def scatter_ones(bidx):
    scps = [pltpu.async_copy(onesv, deg_s.at[bidx.at[j]], semsc, add=True)
            for j in range(EBD)]
    for cp in scps:
      cp.wait()

  fire_cols(0, bidxA, semiA)

  def dpair(bp, carry):
    fire_cols(2 * bp + 1, bidxB, semiB)
    wait_cols(bidxA, semiA)
    scatter_ones(bidxA)
    fire_cols(jnp.minimum(2 * bp + 2, nb - 1), bidxA, semiA)
    wait_cols(bidxB, semiB)
    scatter_ones(bidxB)
    return carry
  lax.fori_loop(0, nb // 2, dpair, 0)

  wait_cols(bidxA, semiA)

  @pl.when(nb % 2 == 1)
  def _():
    scatter_ones(bidxA)

  plsc.subcore_barrier()

  def echunk(aref, lref, ngroups):
    for g in range(ngroups):
      acts = aref[pl.ds(L * g, L)] * H
      locs = lref[pl.ds(L * g, L)] * H
      lofs = (iotav + (L * g)) * H
      dv = _rsqrt_nr(degv[pl.ds(L * g, L)] + 1.0)

      def f4body(f4, carry2):
        for u in range(4):
          f = f4 * 4 + u
          fsp = jnp.full((L,), f, i32)
          va = plsc.load_gather(eav, [acts + fsp])
          vl = plsc.load_gather(elv, [locs + fsp])
          v = jnp.maximum(va + vl, 0.0) * dv
          flat = lofs + fsp
          plsc.store_scatter(
              xbuf2,
              [lax.shift_right_logical(flat, 7),
               jnp.bitwise_and(flat, 127)], v)
        return carry2
      lax.fori_loop(0, H // 4, f4body, 0)

  def ebody(k, carry):
    base = (w + NW * k) * CH
    cp0 = pltpu.async_copy(act_h.at[pl.ds(base, CH)], aidx, sem0)
    cp1 = pltpu.async_copy(loc_h.at[pl.ds(base, CH)], lidx, sem1)
    cp2 = pltpu.async_copy(batch_h.at[pl.ds(base, CH)], bat, sem2)
    cp3 = pltpu.async_copy(deg_s.at[pl.ds(base, CH)], degv, sem3)
    cp0.wait()
    cp1.wait()
    cp3.wait()
    echunk(aidx, lidx, CH // L)
    pltpu.sync_copy(xbuf2, x2_h.at[pl.ds(base // 4, CH * H // 128)])
    cp2.wait()
    pltpu.sync_copy(onesv, cnt_s.at[bat], add=True)
    return carry
  lax.fori_loop(0, _split(NFULL, NW, w), ebody, 0)

  @pl.when(w == NW - 1)
  def _():
    base = NFULL * CH
    pltpu.sync_copy(act_h.at[pl.ds(base, NTAIL)], a32)
    pltpu.sync_copy(loc_h.at[pl.ds(base, NTAIL)], l32)
    pltpu.sync_copy(batch_h.at[pl.ds(base, NTAIL)], b32)
    pltpu.sync_copy(deg_s.at[pl.ds(base, NTAIL)], degv.at[pl.ds(0, NTAIL)])
    echunk(a32, l32, NTAIL // L)
    pltpu.sync_copy(xbuf2.at[pl.ds(0, NTAIL * H // 128)],
                    x2_h.at[pl.ds(base // 4, NTAIL * H // 128)])
    pltpu.sync_copy(ones32, cnt_s.at[b32], add=True)

  plsc.subcore_barrier()

  WB = 6256

  @pl.when(c == 0)
  def _():
    @pl.when(s < NS - 1)
    def _():
      base = s * WB
      pltpu.sync_copy(deg_s.at[pl.ds(base, WB)], deg_h.at[pl.ds(base, WB)])

    @pl.when(s == NS - 1)
    def _():
      base = (NS - 1) * WB
      rem = N - base
      pltpu.sync_copy(deg_s.at[pl.ds(base, rem)], deg_h.at[pl.ds(base, rem)])

  @pl.when(s == 0)
  def _():
    pltpu.sync_copy(cnt_s, cnt_h.at[c])


_sc_embed_deg = pl.kernel(
    _sc_embed_deg_body,
    out_type=(
        jax.ShapeDtypeStruct((XROWS, 128), f32),
        jax.ShapeDtypeStruct((N,), f32),
        jax.ShapeDtypeStruct((2, G), f32),
    ),
    mesh=plsc.VectorSubcoreMesh(core_axis_name="c", subcore_axis_name="s"),
    compiler_params=pltpu.CompilerParams(use_tc_tiling_on_sc=False,
                                         needs_layout_passes=False),
    scratch_types=[
        pltpu.VMEM((CH,), i32),
        pltpu.VMEM((CH,), i32),
        pltpu.VMEM((EBD, CH), i32),
        pltpu.VMEM((EBD, CH), i32),
        pltpu.VMEM((CH,), i32),
        pltpu.VMEM((NTAIL,), i32),
        pltpu.VMEM((NTAIL,), i32),
        pltpu.VMEM((NTAIL,), i32),
        pltpu.VMEM((CH,), f32),
        pltpu.VMEM((CH * H // 128, 128), f32),
        pltpu.VMEM((CH,), f32),
        pltpu.VMEM((NTAIL,), f32),
        pltpu.VMEM((CH,), f32),
        pltpu.VMEM((VA * H,), f32),
        pltpu.VMEM((VL * H,), f32),
        pltpu.VMEM_SHARED((N,), f32),
        pltpu.VMEM_SHARED((G,), f32),
    ] + [pltpu.SemaphoreType.DMA] * 7,
)


_BX = 5000


def _tc_matmul_body(x_ref, w_ref, y_ref):
  y_ref[...] = jnp.dot(x_ref[...], w_ref[...], preferred_element_type=f32)


def _tc_matmul(x2, wbd):
  return pl.pallas_call(
      _tc_matmul_body,
      grid=(XROWS // _BX,),
      in_specs=[
          pl.BlockSpec((_BX, 128), lambda i: (i, 0)),
          pl.BlockSpec((128, 128), lambda i: (0, 0)),
      ],
      out_specs=pl.BlockSpec((_BX, 128), lambda i: (i, 0)),
      out_shape=jax.ShapeDtypeStruct((XROWS, 128), f32),
  )(x2, wbd)


_NRING = 5


def _sc_edge_pool_body(y2_h, deg_h, ei3_h, batch_h, b2_h,
                       out_h,
                       ridxA, cidxA, gidxA, ridxB, cidxB,
                       eidxA, eidxB,
                       g0, g1, g2, g3, g4,
                       abufA, ybufA, degvA, btvA,
                       abufB, ybufB, degvB, btvB,
                       hbuf, bt32, bvv, dinvv,
                       acc_s, pooled_s,
                       semg, semsc, semiA, semiB,
                       semaA, semyA, semdA, sembA,
                       semaB, semyB, semdB, sembB):
  c = lax.axis_index("c")
  s = lax.axis_index("s")
  gbufs = (g0, g1, g2, g3, g4)
  iotav = lax.iota(i32, L)

  def zb_body(i, carry):
    hbuf[i, :] = jnp.zeros((L,), f32)
    return carry
  lax.fori_loop(0, CH, zb_body, 0)

  def zacc(k, carry):
    base = (s + NS * k) * CH
    pltpu.sync_copy(hbuf, acc_s.at[pl.ds(base, CH)])
    return carry
  lax.fori_loop(0, _split(NFULL, NS, s), zacc, 0)

  @pl.when(s == NS - 1)
  def _():
    pltpu.sync_copy(hbuf.at[pl.ds(0, NTAIL)],
                    acc_s.at[pl.ds(NFULL * CH, NTAIL)])

  @pl.when(s == 0)
  def _():
    pltpu.sync_copy(hbuf, pooled_s)

  plsc.subcore_barrier()

  nb = _split(NBLK, NS, s)

  def fire_idx(bk, rdst, cdst, sem):
    blk = (s + NS * bk) * EB
    cp0 = pltpu.async_copy(ei3_h.at[0, pl.ds(blk, EB)], rdst, sem)
    cp1 = pltpu.async_copy(ei3_h.at[1, pl.ds(blk, EB)], cdst, sem)
    return cp0, cp1

  def wait_idx(rdst, cdst, sem):
    pltpu.make_async_copy(ei3_h.at[0, pl.ds(0, EB)], rdst, sem).wait()
    pltpu.make_async_copy(ei3_h.at[1, pl.ds(0, EB)], cdst, sem).wait()

  def process_block(ridx2, cidx2, gidx2):
    for j in range(_NRING):
      for t in range(CH // L):
        v = ridx2[j, pl.ds(L * t, L)]
        gidx2[j, pl.ds(L * t, L)] = v + v + c
    gcps = [None] * EB
    for j in range(_NRING):
      gcps[j] = pltpu.async_copy(y2_h.at[gidx2.at[j]], gbufs[j], semg)
    for j in range(_NRING, EB):
      for t in range(CH // L):
        v = ridx2[j, pl.ds(L * t, L)]
        gidx2[j, pl.ds(L * t, L)] = v + v + c
    scps = [None] * EB
    for j in range(EB):
      gcps[j].wait()
      scps[j] = pltpu.async_copy(gbufs[j % _NRING], acc_s.at[cidx2.at[j]],
                                 semsc, add=True)
      if j + _NRING < EB:
        scps[j].wait()
        gcps[j + _NRING] = pltpu.async_copy(y2_h.at[gidx2.at[j + _NRING]],
                                            gbufs[j % _NRING], semg)
    for j in range(EB - _NRING, EB):
      scps[j].wait()

  fire_idx(0, ridxA, cidxA, semiA)

  def epair(bp, carry):
    bk1 = 2 * bp + 1
    fire_idx(bk1, ridxB, cidxB, semiB)
    wait_idx(ridxA, cidxA, semiA)
    process_block(ridxA, cidxA, gidxA)
    fire_idx(jnp.minimum(2 * bp + 2, nb - 1), ridxA, cidxA, semiA)
    wait_idx(ridxB, cidxB, semiB)
    process_block(ridxB, cidxB, gidxA)
    return carry
  lax.fori_loop(0, nb // 2, epair, 0)

  wait_idx(ridxA, cidxA, semiA)

  @pl.when(nb % 2 == 1)
  def _():
    process_block(ridxA, cidxA, gidxA)

  plsc.subcore_barrier()

  pltpu.sync_copy(b2_h.at[c], bvv)
  bv = bvv[...]
  nch = _split(NFULL, NS, s)

  def build_eidx(eidx, base, nrows):
    for t in range(nrows // L):
      v = base + iotav + (L * t)
      eidx[pl.ds(L * t, L)] = v + v + c

  def fire_chunk(k, eidx, abuf, ybuf, degv, btv, sems):
    base = (s + NS * k) * CH
    build_eidx(eidx, base, CH)
    pltpu.async_copy(acc_s.at[pl.ds(base, CH)], abuf, sems[0])
    pltpu.async_copy(y2_h.at[eidx], ybuf, sems[1])
    pltpu.async_copy(deg_h.at[pl.ds(base, CH)], degv, sems[2])
    pltpu.async_copy(batch_h.at[pl.ds(base, CH)], btv, sems[3])

  def wait_chunk(abuf, ybuf, degv, btv, sems):
    pltpu.make_async_copy(y2_h.at[pl.ds(0, CH)], abuf, sems[0]).wait()
    pltpu.make_async_copy(y2_h.at[pl.ds(0, CH)], ybuf, sems[1]).wait()
    pltpu.make_async_copy(deg_h.at[pl.ds(0, CH)], degv, sems[2]).wait()
    pltpu.make_async_copy(batch_h.at[pl.ds(0, CH)], btv, sems[3]).wait()

  def compute_chunk(abuf, ybuf, degv, btv, nrows):
    for t in range(nrows // L):
      d = degv[pl.ds(L * t, L)] + 1.0
      dinvv[pl.ds(L * t, L)] = _rsqrt_nr(d)

    def hrow(i, carry2):
      for u in range(4):
        ii = i * 4 + u
        dsp = plsc.load_gather(dinvv, [jnp.full((L,), ii, i32)])
        v = (abuf[ii, :] + ybuf[ii, :]) * dsp + bv
        hbuf[ii, :] = jnp.maximum(v, 0.0)
      return carry2
    lax.fori_loop(0, nrows // 4, hrow, 0)
    pltpu.sync_copy(hbuf.at[pl.ds(0, nrows)], pooled_s.at[btv], add=True)

  semsA = (semaA, semyA, semdA, sembA)
  semsB = (semaB, semyB, semdB, sembB)
  fire_chunk(0, eidxA, abufA, ybufA, degvA, btvA, semsA)

  def ppair(kp, carry):
    fire_chunk(2 * kp + 1, eidxB, abufB, ybufB, degvB, btvB, semsB)
    wait_chunk(abufA, ybufA, degvA, btvA, semsA)
    compute_chunk(abufA, ybufA, degvA, btvA, CH)
    fire_chunk(jnp.minimum(2 * kp + 2, nch - 1), eidxA, abufA, ybufA,
               degvA, btvA, semsA)
    wait_chunk(abufB, ybufB, degvB, btvB, semsB)
    compute_chunk(abufB, ybufB, degvB, btvB, CH)
    return carry
  lax.fori_loop(0, nch // 2, ppair, 0)

  wait_chunk(abufA, ybufA, degvA, btvA, semsA)

  @pl.when(nch % 2 == 1)
  def _():
    compute_chunk(abufA, ybufA, degvA, btvA, CH)

  @pl.when(s == NS - 1)
  def _():
    base = NFULL * CH
    build_eidx(eidxA, base, NTAIL)
    pltpu.sync_copy(acc_s.at[pl.ds(base, NTAIL)], abufA.at[pl.ds(0, NTAIL)])
    cpy = pltpu.async_copy(y2_h.at[eidxA.at[pl.ds(0, NTAIL)]],
                           ybufA.at[pl.ds(0, NTAIL)], semyA)
    pltpu.sync_copy(deg_h.at[pl.ds(base, NTAIL)], degvA.at[pl.ds(0, NTAIL)])
    pltpu.sync_copy(batch_h.at[pl.ds(base, NTAIL)], bt32)
    cpy.wait()
    for t in range(NTAIL // L):
      d = degvA[pl.ds(L * t, L)] + 1.0
      dinvv[pl.ds(L * t, L)] = _rsqrt_nr(d)

    def hrow32(i, carry2):
      for u in range(4):
        ii = i * 4 + u
        dsp = plsc.load_gather(dinvv, [jnp.full((L,), ii, i32)])
        v = (abufA[ii, :] + ybufA[ii, :]) * dsp + bv
        hbuf[ii, :] = jnp.maximum(v, 0.0)
      return carry2
    lax.fori_loop(0, NTAIL // 4, hrow32, 0)
    pltpu.sync_copy(hbuf.at[pl.ds(0, NTAIL)], pooled_s.at[bt32], add=True)

  plsc.subcore_barrier()

  @pl.when(s == 0)
  def _():
    pltpu.sync_copy(pooled_s, out_h.at[c])


_sc_edge_pool = pl.kernel(
    _sc_edge_pool_body,
    out_type=jax.ShapeDtypeStruct((2, G, L), f32),
    mesh=plsc.VectorSubcoreMesh(core_axis_name="c", subcore_axis_name="s"),
    compiler_params=pltpu.CompilerParams(use_tc_tiling_on_sc=False,
                                         needs_layout_passes=False),
    scratch_types=[
        pltpu.VMEM((EB, CH), i32),
        pltpu.VMEM((EB, CH), i32),
        pltpu.VMEM((EB, CH), i32),
        pltpu.VMEM((EB, CH), i32),
        pltpu.VMEM((EB, CH), i32),
        pltpu.VMEM((CH,), i32),
        pltpu.VMEM((CH,), i32),
    ] + [pltpu.VMEM((CH, L), f32)] * _NRING + [
        pltpu.VMEM((CH, L), f32),
        pltpu.VMEM((CH, L), f32),
        pltpu.VMEM((CH,), f32),
        pltpu.VMEM((CH,), i32),
        pltpu.VMEM((CH, L), f32),
        pltpu.VMEM((CH, L), f32),
        pltpu.VMEM((CH,), f32),
        pltpu.VMEM((CH,), i32),
        pltpu.VMEM((CH, L), f32),
        pltpu.VMEM((NTAIL,), i32),
        pltpu.VMEM((L,), f32),
        pltpu.VMEM((CH,), f32),
        pltpu.VMEM_SHARED((N, L), f32),
        pltpu.VMEM_SHARED((G, L), f32),
    ] + [pltpu.SemaphoreType.DMA] * 12,
)


def _tc_head_body(p_ref, cnt_ref, wfc_ref, bfc_ref, out_ref):
  ps = jnp.concatenate([p_ref[0], p_ref[1]], axis=1)
  cnt = jnp.maximum(cnt_ref[0] + cnt_ref[1], 1.0)
  pooled = ps / cnt
  logits = jnp.dot(pooled, wfc_ref[...], preferred_element_type=f32)
  logits = logits + bfc_ref[...]
  m = jnp.max(logits, axis=1, keepdims=True)
  e = jnp.exp(logits - m)
  lse = jnp.log(jnp.sum(e, axis=1, keepdims=True)) + m
  out_ref[...] = logits - lse


def _tc_head(pooled, cnt3, wfc, bfc2):
  return pl.pallas_call(
      _tc_head_body,
      out_shape=jax.ShapeDtypeStruct((G, T), f32),
  )(pooled, cnt3, wfc, bfc2)


def kernel(act, location, edge_index, batch, emb_act, emb_loc,
           W_gcn, b_gcn, W_fc, b_fc):
  act = act.astype(i32)
  location = location.astype(i32)
  edge_index = edge_index.astype(i32)
  batch = batch.astype(i32)
  ei3 = edge_index.reshape(2, ECH, CH)
  wbd = jnp.kron(jnp.eye(4, dtype=f32), W_gcn)

  x2, deg, cnt = _sc_embed_deg(act, location, ei3, batch,
                               emb_act.reshape(VA * H),
                               emb_loc.reshape(VL * H))
  y2 = _tc_matmul(x2, wbd)
  pooled = _sc_edge_pool(y2.reshape(2 * N, L), deg, ei3, batch,
                         b_gcn.reshape(2, L))
  return _tc_head(pooled, cnt.reshape(2, G, 1), W_fc, b_fc.reshape(1, T))
